# Initial kernel scaffold; baseline (speedup 1.0000x reference)
import jax
import jax.numpy as jnp
from jax.experimental import pallas as pl


def kernel(x, edge_index, batch, Wl1, Wr1, att1, b1, Wl2, Wr2, att2, b2, Wlin, blin):
    raise NotImplementedError("write your pallas kernel here")


# speedup vs baseline: 13.6498x; 13.6498x over previous
import functools

import jax
import jax.numpy as jnp
from jax import lax
from jax.experimental import pallas as pl
from jax.experimental.pallas import tpu as pltpu
from jax.experimental.pallas import tpu_sc as plsc

N = 10000
NPAD = 10240
F_IN = 128
HID = 128
NC = 10
NG = 16
E_TOT = 320000 + N
EB = 96
NBLK = 3456
EPAD = NBLK * EB
BLK_T1 = NBLK // 16
BLK_W2 = NBLK // 32
ROWS_T = NPAD // 16
RB = 1280
NRB = NPAD // RB


def _proj1_body(x_ref, wl_ref, wr_ref, xl_ref, xr_ref):
    x = x_ref[...]
    xl_ref[...] = jnp.dot(x, wl_ref[...], preferred_element_type=jnp.float32)
    xr_ref[...] = jnp.dot(x, wr_ref[...], preferred_element_type=jnp.float32)


def _proj1(xpad, Wl1, Wr1):
    return pl.pallas_call(
        _proj1_body,
        grid=(NRB, 2),
        in_specs=[
            pl.BlockSpec((RB, F_IN), lambda i, h: (i, 0)),
            pl.BlockSpec((F_IN, HID), lambda i, h: (0, h)),
            pl.BlockSpec((F_IN, HID), lambda i, h: (0, h)),
        ],
        out_specs=[
            pl.BlockSpec((RB, HID), lambda i, h: (h * NRB + i, 0)),
            pl.BlockSpec((RB, HID), lambda i, h: (h * NRB + i, 0)),
        ],
        out_shape=[
            jax.ShapeDtypeStruct((2 * NPAD, HID), jnp.float32),
            jax.ShapeDtypeStruct((2 * NPAD, HID), jnp.float32),
        ],
    )(xpad, Wl1, Wr1)


def _elu(v):
    return jnp.where(v > 0, v, jnp.exp(v) - 1.0)


def _mid_body(h_ref, b1_ref, wl_ref, wr_ref, xl_ref, xr_ref):
    e0 = _elu(h_ref[0] + b1_ref[0])
    e1 = _elu(h_ref[1] + b1_ref[1])
    wl = wl_ref[...]
    wr = wr_ref[...]
    xl_ref[...] = (jnp.dot(e0, wl[:HID], preferred_element_type=jnp.float32)
                   + jnp.dot(e1, wl[HID:], preferred_element_type=jnp.float32))
    xr_ref[...] = (jnp.dot(e0, wr[:HID], preferred_element_type=jnp.float32)
                   + jnp.dot(e1, wr[HID:], preferred_element_type=jnp.float32))


def _mid(h1r, b1r, Wl2, Wr2):
    return pl.pallas_call(
        _mid_body,
        grid=(NRB,),
        in_specs=[
            pl.BlockSpec((2, RB, HID), lambda i: (0, i, 0)),
            pl.BlockSpec((2, HID), lambda i: (0, 0)),
            pl.BlockSpec((2 * HID, HID), lambda i: (0, 0)),
            pl.BlockSpec((2 * HID, HID), lambda i: (0, 0)),
        ],
        out_specs=[
            pl.BlockSpec((RB, HID), lambda i: (i, 0)),
            pl.BlockSpec((RB, HID), lambda i: (i, 0)),
        ],
        out_shape=[
            jax.ShapeDtypeStruct((NPAD, HID), jnp.float32),
            jax.ShapeDtypeStruct((NPAD, HID), jnp.float32),
        ],
    )(h1r, b1r, Wl2, Wr2)


def _final_body(p_ref, b2_ref, batch_ref, wlin_ref, blin_ref, out_ref,
                sum_scr, cnt_scr):
    i = pl.program_id(0)

    @pl.when(i == 0)
    def _():
        sum_scr[...] = jnp.zeros((NG, HID), jnp.float32)
        cnt_scr[...] = jnp.zeros((NG, HID), jnp.float32)

    h = _elu(p_ref[0] + p_ref[1] + b2_ref[0])
    b = batch_ref[0, 0, :]
    P = (lax.broadcasted_iota(jnp.int32, (NG, RB), 0) == b[None, :]
         ).astype(jnp.float32)
    sum_scr[...] += jnp.dot(P, h, preferred_element_type=jnp.float32)
    cnt_scr[...] += jnp.dot(P, jnp.ones((RB, HID), jnp.float32),
                            preferred_element_type=jnp.float32)

    @pl.when(i == NRB - 1)
    def _():
        pooled = sum_scr[...] / jnp.maximum(cnt_scr[...], 1.0)
        out_ref[...] = (jnp.dot(pooled, wlin_ref[...],
                                preferred_element_type=jnp.float32)
                        + blin_ref[0])


def _final(p2r, b2r, batch3, wlin_p, blin_p):
    return pl.pallas_call(
        _final_body,
        grid=(NRB,),
        in_specs=[
            pl.BlockSpec((2, RB, HID), lambda i: (0, i, 0)),
            pl.BlockSpec((1, HID), lambda i: (0, 0)),
            pl.BlockSpec((1, 1, RB), lambda i: (i, 0, 0)),
            pl.BlockSpec((HID, 128), lambda i: (0, 0)),
            pl.BlockSpec((1, 128), lambda i: (0, 0)),
        ],
        out_specs=pl.BlockSpec((NG, 128), lambda i: (0, 0)),
        out_shape=jax.ShapeDtypeStruct((NG, 128), jnp.float32),
        scratch_shapes=[
            pltpu.VMEM((NG, HID), jnp.float32),
            pltpu.VMEM((NG, HID), jnp.float32),
        ],
    )(p2r, b2r, batch3, wlin_p, blin_p)


_MESH = plsc.VectorSubcoreMesh(core_axis_name="c", subcore_axis_name="s")


def _edge_block_logits(xls_v, xrd_v, att_vecs, ebuf, e_v):
    def edge_body(je, carry):
        acc = jnp.zeros((16,), jnp.float32)
        for k in range(HID // 16):
            a = xls_v[je, pl.ds(16 * k, 16)]
            b = xrd_v[je, pl.ds(16 * k, 16)]
            h = a + b
            h = jnp.where(h > 0, h, 0.2 * h)
            acc = acc + h * att_vecs[k]
        ebuf[je, pl.ds(0, 16)] = acc
        return carry

    lax.fori_loop(0, EB, edge_body, 0)
    iota = lax.broadcasted_iota(jnp.int32, (16,), 0)
    for g in range(EB // 16):
        rows = iota + (16 * g)
        tot = jnp.zeros((16,), jnp.float32)
        for l in range(16):
            tot = tot + plsc.load_gather(
                ebuf, [rows, jnp.full((16,), l, jnp.int32)])
        e_v[pl.ds(16 * g, 16)] = jnp.exp(tot)


def _scale_rows(xls_v, a_v):
    def body(je, carry):
        a = a_v[pl.ds(je, 16)][0]
        for k in range(HID // 16):
            xls_v[je, pl.ds(16 * k, 16)] = xls_v[je, pl.ds(16 * k, 16)] * a
        return carry

    lax.fori_loop(0, EB, body, 0)


@functools.partial(
    pl.kernel,
    out_type=[jax.ShapeDtypeStruct((2 * NPAD, HID), jnp.float32),
              jax.ShapeDtypeStruct((2 * EPAD,), jnp.float32)],
    mesh=_MESH,
    compiler_params=pltpu.CompilerParams(needs_layout_passes=False),
    scratch_types=[
        pltpu.VMEM((EB,), jnp.int32),
        pltpu.VMEM((EB,), jnp.int32),
        pltpu.VMEM((EB,), jnp.int32),
        pltpu.VMEM((EB, HID), jnp.float32),
        pltpu.VMEM((EB, HID), jnp.float32),
        pltpu.VMEM((EB,), jnp.float32),
        pltpu.VMEM((EB + 16,), jnp.float32),
        pltpu.VMEM((EB,), jnp.float32),
        pltpu.VMEM((HID,), jnp.float32),
        pltpu.VMEM((EB, 17), jnp.float32),
        pltpu.VMEM_SHARED((NPAD,), jnp.float32),
        pltpu.VMEM_SHARED((NPAD, HID), jnp.float32),
    ],
)
def _gat1_sc(xl_hbm, xr_hbm, src_hbm, dst_hbm, att_hbm, z1_hbm, z2_hbm,
             out_hbm, w_hbm, src_v, dst_v, gidx_v, xls_v, xrd_v, e_v, a_v,
             dd_v, att_v, ebuf, denom_sh, out_sh):
    c = lax.axis_index("c")
    s = lax.axis_index("s")
    noff = c * NPAD
    r0 = s * ROWS_T

    pltpu.sync_copy(z1_hbm.at[pl.ds(r0, ROWS_T)],
                    denom_sh.at[pl.ds(r0, ROWS_T)])
    pltpu.sync_copy(z2_hbm.at[pl.ds(r0, ROWS_T)],
                    out_sh.at[pl.ds(r0, ROWS_T)])
    pltpu.sync_copy(att_hbm.at[c], att_v)
    plsc.subcore_barrier()

    att_vecs = [att_v[pl.ds(16 * k, 16)] for k in range(HID // 16)]

    def passA(bi, carry):
        base = (s * BLK_T1 + bi) * EB
        pltpu.sync_copy(src_hbm.at[pl.ds(base, EB)], src_v)
        pltpu.sync_copy(dst_hbm.at[pl.ds(base, EB)], dst_v)
        for g in range(EB // 16):
            gidx_v[pl.ds(16 * g, 16)] = src_v[pl.ds(16 * g, 16)] + noff
        pltpu.sync_copy(xl_hbm.at[gidx_v], xls_v)
        for g in range(EB // 16):
            gidx_v[pl.ds(16 * g, 16)] = dst_v[pl.ds(16 * g, 16)] + noff
        pltpu.sync_copy(xr_hbm.at[gidx_v], xrd_v)
        _edge_block_logits(xls_v, xrd_v, att_vecs, ebuf, e_v)
        pltpu.sync_copy(e_v, w_hbm.at[pl.ds(c * EPAD + base, EB)])
        pltpu.sync_copy(e_v, denom_sh.at[dst_v], add=True)
        return carry

    lax.fori_loop(0, BLK_T1, passA, 0)
    plsc.subcore_barrier()

    def passB(bi, carry):
        base = (s * BLK_T1 + bi) * EB
        pltpu.sync_copy(src_hbm.at[pl.ds(base, EB)], src_v)
        pltpu.sync_copy(dst_hbm.at[pl.ds(base, EB)], dst_v)
        for g in range(EB // 16):
            gidx_v[pl.ds(16 * g, 16)] = src_v[pl.ds(16 * g, 16)] + noff
        pltpu.sync_copy(xl_hbm.at[gidx_v], xls_v)
        pltpu.sync_copy(w_hbm.at[pl.ds(c * EPAD + base, EB)], e_v)
        pltpu.sync_copy(denom_sh.at[dst_v], dd_v)
        for g in range(EB // 16):
            a_v[pl.ds(16 * g, 16)] = (e_v[pl.ds(16 * g, 16)]
                                      / (dd_v[pl.ds(16 * g, 16)] + 1e-16))
        _scale_rows(xls_v, a_v)
        pltpu.sync_copy(xls_v, out_sh.at[dst_v], add=True)
        return carry

    lax.fori_loop(0, BLK_T1, passB, 0)
    plsc.subcore_barrier()
    pltpu.sync_copy(out_sh.at[pl.ds(r0, ROWS_T)],
                    out_hbm.at[pl.ds(noff + r0, ROWS_T)])


@functools.partial(
    pl.kernel,
    out_type=[jax.ShapeDtypeStruct((2 * NPAD, HID), jnp.float32),
              jax.ShapeDtypeStruct((EPAD,), jnp.float32)],
    mesh=_MESH,
    compiler_params=pltpu.CompilerParams(needs_layout_passes=False),
    scratch_types=[
        pltpu.VMEM((EB,), jnp.int32),
        pltpu.VMEM((EB,), jnp.int32),
        pltpu.VMEM((EB, HID), jnp.float32),
        pltpu.VMEM((EB, HID), jnp.float32),
        pltpu.VMEM((EB,), jnp.float32),
        pltpu.VMEM((EB + 16,), jnp.float32),
        pltpu.VMEM((EB,), jnp.float32),
        pltpu.VMEM((HID,), jnp.float32),
        pltpu.VMEM((EB, 17), jnp.float32),
        pltpu.VMEM_SHARED((NPAD,), jnp.float32),
        pltpu.VMEM_SHARED((NPAD, HID), jnp.float32),
    ],
)
def _gat2_sc(xl_hbm, xr_hbm, src_hbm, dst_hbm, att_hbm, z1_hbm, z2_hbm,
             out_hbm, w_hbm, src_v, dst_v, xls_v, xrd_v, e_v, a_v, dd_v,
             att_v, ebuf, denom_sh, out_sh):
    c = lax.axis_index("c")
    s = lax.axis_index("s")
    r0 = s * ROWS_T

    pltpu.sync_copy(z1_hbm.at[pl.ds(r0, ROWS_T)],
                    denom_sh.at[pl.ds(r0, ROWS_T)])
    pltpu.sync_copy(z2_hbm.at[pl.ds(r0, ROWS_T)],
                    out_sh.at[pl.ds(r0, ROWS_T)])
    pltpu.sync_copy(att_hbm.at[0], att_v)
    plsc.subcore_barrier()

    att_vecs = [att_v[pl.ds(16 * k, 16)] for k in range(HID // 16)]
    my_blk0 = (c * 16 + s) * BLK_W2
    other_blk0 = ((1 - c) * 16 + s) * BLK_W2

    def passA(blk0, save_w):
        def body(bi, carry):
            base = (blk0 + bi) * EB
            pltpu.sync_copy(src_hbm.at[pl.ds(base, EB)], src_v)
            pltpu.sync_copy(dst_hbm.at[pl.ds(base, EB)], dst_v)
            pltpu.sync_copy(xl_hbm.at[src_v], xls_v)
            pltpu.sync_copy(xr_hbm.at[dst_v], xrd_v)
            _edge_block_logits(xls_v, xrd_v, att_vecs, ebuf, e_v)
            if save_w:
                pltpu.sync_copy(e_v, w_hbm.at[pl.ds(base, EB)])
            pltpu.sync_copy(e_v, denom_sh.at[dst_v], add=True)
            return carry

        lax.fori_loop(0, BLK_W2, body, 0)

    passA(my_blk0, True)
    passA(other_blk0, False)
    plsc.subcore_barrier()

    def passB(bi, carry):
        base = (my_blk0 + bi) * EB
        pltpu.sync_copy(src_hbm.at[pl.ds(base, EB)], src_v)
        pltpu.sync_copy(dst_hbm.at[pl.ds(base, EB)], dst_v)
        pltpu.sync_copy(xl_hbm.at[src_v], xls_v)
        pltpu.sync_copy(w_hbm.at[pl.ds(base, EB)], e_v)
        pltpu.sync_copy(denom_sh.at[dst_v], dd_v)
        for g in range(EB // 16):
            a_v[pl.ds(16 * g, 16)] = (e_v[pl.ds(16 * g, 16)]
                                      / (dd_v[pl.ds(16 * g, 16)] + 1e-16))
        _scale_rows(xls_v, a_v)
        pltpu.sync_copy(xls_v, out_sh.at[dst_v], add=True)
        return carry

    lax.fori_loop(0, BLK_W2, passB, 0)
    plsc.subcore_barrier()
    pltpu.sync_copy(out_sh.at[pl.ds(r0, ROWS_T)],
                    out_hbm.at[pl.ds(c * NPAD + r0, ROWS_T)])


def kernel(x, edge_index, batch, Wl1, Wr1, att1, b1, Wl2, Wr2, att2, b2,
           Wlin, blin):
    f32 = jnp.float32
    i32 = jnp.int32
    npad_e = EPAD - E_TOT
    loops = jnp.arange(N, dtype=i32)
    pad_src = jnp.arange(npad_e, dtype=i32) % N
    pad_dst = N + jnp.arange(npad_e, dtype=i32) % (NPAD - N)
    src = jnp.concatenate([edge_index[0].astype(i32), loops, pad_src])
    dst = jnp.concatenate([edge_index[1].astype(i32), loops, pad_dst])

    xpad = jnp.pad(x.astype(f32), ((0, NPAD - N), (0, 0)))
    z1 = jnp.zeros((NPAD,), f32)
    z2 = jnp.zeros((NPAD, HID), f32)

    xl1, xr1 = _proj1(xpad, Wl1, Wr1)
    h1, _ = _gat1_sc(xl1, xr1, src, dst, att1, z1, z2)
    xl2, xr2 = _mid(h1.reshape(2, NPAD, HID), b1.reshape(2, HID), Wl2, Wr2)
    h2, _ = _gat2_sc(xl2, xr2, src, dst, att2, z1, z2)

    batch3 = jnp.concatenate(
        [batch.astype(i32), jnp.full((NPAD - N,), NG, i32)]).reshape(NRB, 1, RB)
    wlin_p = jnp.pad(Wlin.astype(f32), ((0, 0), (0, 128 - NC)))
    blin_p = jnp.pad(blin.astype(f32), (0, 128 - NC)).reshape(1, 128)
    logits = _final(h2.reshape(2, NPAD, HID), b2.reshape(1, HID), batch3,
                    wlin_p, blin_p)
    return logits[:, :NC]


# speedup vs baseline: 18.8428x; 1.3805x over previous
import functools

import jax
import jax.numpy as jnp
from jax import lax
from jax.experimental import pallas as pl
from jax.experimental.pallas import tpu as pltpu
from jax.experimental.pallas import tpu_sc as plsc

N = 10000
NPAD = 10240
F_IN = 128
HID = 128
NC = 10
NG = 16
E_TOT = 320000 + N
EB = 96
NBLK = 3456
EPAD = NBLK * EB
BLK_T1 = NBLK // 16
BLK_W2 = NBLK // 32
ROWS_T = NPAD // 16
RB = 1280
NRB = NPAD // RB


def _proj1_body(x_ref, wl_ref, wr_ref, xl_ref, xr_ref):
    x = x_ref[...]
    xl_ref[...] = jnp.dot(x, wl_ref[...], preferred_element_type=jnp.float32)
    xr_ref[...] = jnp.dot(x, wr_ref[...], preferred_element_type=jnp.float32)


def _proj1(xpad, Wl1, Wr1):
    return pl.pallas_call(
        _proj1_body,
        grid=(NRB, 2),
        in_specs=[
            pl.BlockSpec((RB, F_IN), lambda i, h: (i, 0)),
            pl.BlockSpec((F_IN, HID), lambda i, h: (0, h)),
            pl.BlockSpec((F_IN, HID), lambda i, h: (0, h)),
        ],
        out_specs=[
            pl.BlockSpec((RB, HID), lambda i, h: (h * NRB + i, 0)),
            pl.BlockSpec((RB, HID), lambda i, h: (h * NRB + i, 0)),
        ],
        out_shape=[
            jax.ShapeDtypeStruct((2 * NPAD, HID), jnp.float32),
            jax.ShapeDtypeStruct((2 * NPAD, HID), jnp.float32),
        ],
    )(xpad, Wl1, Wr1)


def _elu(v):
    return jnp.where(v > 0, v, jnp.exp(v) - 1.0)


def _mid_body(h_ref, b1_ref, wl_ref, wr_ref, xl_ref, xr_ref):
    e0 = _elu(h_ref[0] + b1_ref[0])
    e1 = _elu(h_ref[1] + b1_ref[1])
    wl = wl_ref[...]
    wr = wr_ref[...]
    xl_ref[...] = (jnp.dot(e0, wl[:HID], preferred_element_type=jnp.float32)
                   + jnp.dot(e1, wl[HID:], preferred_element_type=jnp.float32))
    xr_ref[...] = (jnp.dot(e0, wr[:HID], preferred_element_type=jnp.float32)
                   + jnp.dot(e1, wr[HID:], preferred_element_type=jnp.float32))


def _mid(h1r, b1r, Wl2, Wr2):
    return pl.pallas_call(
        _mid_body,
        grid=(NRB,),
        in_specs=[
            pl.BlockSpec((2, RB, HID), lambda i: (0, i, 0)),
            pl.BlockSpec((2, HID), lambda i: (0, 0)),
            pl.BlockSpec((2 * HID, HID), lambda i: (0, 0)),
            pl.BlockSpec((2 * HID, HID), lambda i: (0, 0)),
        ],
        out_specs=[
            pl.BlockSpec((RB, HID), lambda i: (i, 0)),
            pl.BlockSpec((RB, HID), lambda i: (i, 0)),
        ],
        out_shape=[
            jax.ShapeDtypeStruct((NPAD, HID), jnp.float32),
            jax.ShapeDtypeStruct((NPAD, HID), jnp.float32),
        ],
    )(h1r, b1r, Wl2, Wr2)


def _final_body(p_ref, b2_ref, batch_ref, wlin_ref, blin_ref, out_ref,
                sum_scr, cnt_scr):
    i = pl.program_id(0)

    @pl.when(i == 0)
    def _():
        sum_scr[...] = jnp.zeros((NG, HID), jnp.float32)
        cnt_scr[...] = jnp.zeros((NG, HID), jnp.float32)

    h = _elu(p_ref[0] + p_ref[1] + b2_ref[0])
    b = batch_ref[0, 0, :]
    P = (lax.broadcasted_iota(jnp.int32, (NG, RB), 0) == b[None, :]
         ).astype(jnp.float32)
    sum_scr[...] += jnp.dot(P, h, preferred_element_type=jnp.float32)
    cnt_scr[...] += jnp.dot(P, jnp.ones((RB, HID), jnp.float32),
                            preferred_element_type=jnp.float32)

    @pl.when(i == NRB - 1)
    def _():
        pooled = sum_scr[...] / jnp.maximum(cnt_scr[...], 1.0)
        out_ref[...] = (jnp.dot(pooled, wlin_ref[...],
                                preferred_element_type=jnp.float32)
                        + blin_ref[0])


def _final(p2r, b2r, batch3, wlin_p, blin_p):
    return pl.pallas_call(
        _final_body,
        grid=(NRB,),
        in_specs=[
            pl.BlockSpec((2, RB, HID), lambda i: (0, i, 0)),
            pl.BlockSpec((1, HID), lambda i: (0, 0)),
            pl.BlockSpec((1, 1, RB), lambda i: (i, 0, 0)),
            pl.BlockSpec((HID, 128), lambda i: (0, 0)),
            pl.BlockSpec((1, 128), lambda i: (0, 0)),
        ],
        out_specs=pl.BlockSpec((NG, 128), lambda i: (0, 0)),
        out_shape=jax.ShapeDtypeStruct((NG, 128), jnp.float32),
        scratch_shapes=[
            pltpu.VMEM((NG, HID), jnp.float32),
            pltpu.VMEM((NG, HID), jnp.float32),
        ],
    )(p2r, b2r, batch3, wlin_p, blin_p)


_MESH = plsc.VectorSubcoreMesh(core_axis_name="c", subcore_axis_name="s")


def _edge_block_logits(xls_v, xrd_v, att_vecs, ebuf, e_v):
    @plsc.parallel_loop(0, EB, 1, unroll=4)
    def edge_body(je):
        acc = jnp.zeros((16,), jnp.float32)
        for k in range(HID // 16):
            a = xls_v[je, pl.ds(16 * k, 16)]
            b = xrd_v[je, pl.ds(16 * k, 16)]
            h = a + b
            h = jnp.where(h > 0, h, 0.2 * h)
            acc = acc + h * att_vecs[k]
        ebuf[je, pl.ds(0, 16)] = acc
    iota = lax.broadcasted_iota(jnp.int32, (16,), 0)
    for g in range(EB // 16):
        rows = iota + (16 * g)
        tot = jnp.zeros((16,), jnp.float32)
        for l in range(16):
            tot = tot + plsc.load_gather(
                ebuf, [rows, jnp.full((16,), l, jnp.int32)])
        e_v[pl.ds(16 * g, 16)] = jnp.exp(tot)


def _scale_rows(xls_v, a_v):
    @plsc.parallel_loop(0, EB, 1, unroll=4)
    def body(je):
        a = a_v[pl.ds(je, 16)][0]
        for k in range(HID // 16):
            xls_v[je, pl.ds(16 * k, 16)] = xls_v[je, pl.ds(16 * k, 16)] * a


@functools.partial(
    pl.kernel,
    out_type=[jax.ShapeDtypeStruct((2 * NPAD, HID), jnp.float32),
              jax.ShapeDtypeStruct((2 * EPAD,), jnp.float32)],
    mesh=_MESH,
    compiler_params=pltpu.CompilerParams(needs_layout_passes=False),
    scratch_types=[
        pltpu.VMEM((EB,), jnp.int32),
        pltpu.VMEM((EB,), jnp.int32),
        pltpu.VMEM((EB,), jnp.int32),
        pltpu.VMEM((EB,), jnp.int32),
        pltpu.VMEM((EB, HID), jnp.float32),
        pltpu.VMEM((EB, HID), jnp.float32),
        pltpu.VMEM((EB,), jnp.float32),
        pltpu.VMEM((EB + 16,), jnp.float32),
        pltpu.VMEM((EB,), jnp.float32),
        pltpu.VMEM((HID,), jnp.float32),
        pltpu.VMEM((EB, 17), jnp.float32),
        pltpu.SemaphoreType.DMA,
        pltpu.SemaphoreType.DMA,
        pltpu.SemaphoreType.DMA,
        pltpu.VMEM_SHARED((NPAD,), jnp.float32),
        pltpu.VMEM_SHARED((NPAD, HID), jnp.float32),
    ],
)
def _gat1_sc(xl_hbm, xr_hbm, src_hbm, dst_hbm, att_hbm, z1_hbm, z2_hbm,
             out_hbm, w_hbm, src_v, dst_v, gidx_v, gidx2_v, xls_v, xrd_v,
             e_v, a_v, dd_v, att_v, ebuf, sem_a, sem_b, sem_c, denom_sh,
             out_sh):
    c = lax.axis_index("c")
    s = lax.axis_index("s")
    noff = c * NPAD
    r0 = s * ROWS_T

    pltpu.sync_copy(z1_hbm.at[pl.ds(r0, ROWS_T)],
                    denom_sh.at[pl.ds(r0, ROWS_T)])
    pltpu.sync_copy(z2_hbm.at[pl.ds(r0, ROWS_T)],
                    out_sh.at[pl.ds(r0, ROWS_T)])
    pltpu.sync_copy(att_hbm.at[c], att_v)
    plsc.subcore_barrier()

    att_vecs = [att_v[pl.ds(16 * k, 16)] for k in range(HID // 16)]

    def passA(bi, carry):
        base = (s * BLK_T1 + bi) * EB
        cps = pltpu.async_copy(src_hbm.at[pl.ds(base, EB)], src_v, sem_a)
        cpd = pltpu.async_copy(dst_hbm.at[pl.ds(base, EB)], dst_v, sem_b)
        cps.wait()
        for g in range(EB // 16):
            gidx_v[pl.ds(16 * g, 16)] = src_v[pl.ds(16 * g, 16)] + noff
        cpl = pltpu.async_copy(xl_hbm.at[gidx_v], xls_v, sem_a)
        cpd.wait()
        for g in range(EB // 16):
            gidx2_v[pl.ds(16 * g, 16)] = dst_v[pl.ds(16 * g, 16)] + noff
        cpr = pltpu.async_copy(xr_hbm.at[gidx2_v], xrd_v, sem_b)
        cpl.wait()
        cpr.wait()
        _edge_block_logits(xls_v, xrd_v, att_vecs, ebuf, e_v)
        pltpu.sync_copy(e_v, w_hbm.at[pl.ds(c * EPAD + base, EB)])
        pltpu.sync_copy(e_v, denom_sh.at[dst_v], add=True)
        return carry

    lax.fori_loop(0, BLK_T1, passA, 0)
    plsc.subcore_barrier()

    def passB(bi, carry):
        base = (s * BLK_T1 + bi) * EB
        cps = pltpu.async_copy(src_hbm.at[pl.ds(base, EB)], src_v, sem_a)
        cpd = pltpu.async_copy(dst_hbm.at[pl.ds(base, EB)], dst_v, sem_b)
        cpw = pltpu.async_copy(w_hbm.at[pl.ds(c * EPAD + base, EB)], e_v,
                               sem_c)
        cps.wait()
        for g in range(EB // 16):
            gidx_v[pl.ds(16 * g, 16)] = src_v[pl.ds(16 * g, 16)] + noff
        cpl = pltpu.async_copy(xl_hbm.at[gidx_v], xls_v, sem_a)
        cpd.wait()
        cpdd = pltpu.async_copy(denom_sh.at[dst_v], dd_v, sem_b)
        cpw.wait()
        cpdd.wait()
        cpl.wait()
        for g in range(EB // 16):
            a_v[pl.ds(16 * g, 16)] = (e_v[pl.ds(16 * g, 16)]
                                      / (dd_v[pl.ds(16 * g, 16)] + 1e-16))
        _scale_rows(xls_v, a_v)
        pltpu.sync_copy(xls_v, out_sh.at[dst_v], add=True)
        return carry

    lax.fori_loop(0, BLK_T1, passB, 0)
    plsc.subcore_barrier()
    pltpu.sync_copy(out_sh.at[pl.ds(r0, ROWS_T)],
                    out_hbm.at[pl.ds(noff + r0, ROWS_T)])


@functools.partial(
    pl.kernel,
    out_type=[jax.ShapeDtypeStruct((2 * NPAD, HID), jnp.float32),
              jax.ShapeDtypeStruct((EPAD,), jnp.float32)],
    mesh=_MESH,
    compiler_params=pltpu.CompilerParams(needs_layout_passes=False),
    scratch_types=[
        pltpu.VMEM((EB,), jnp.int32),
        pltpu.VMEM((EB,), jnp.int32),
        pltpu.VMEM((EB, HID), jnp.float32),
        pltpu.VMEM((EB, HID), jnp.float32),
        pltpu.VMEM((EB,), jnp.float32),
        pltpu.VMEM((EB + 16,), jnp.float32),
        pltpu.VMEM((EB,), jnp.float32),
        pltpu.VMEM((HID,), jnp.float32),
        pltpu.VMEM((EB, 17), jnp.float32),
        pltpu.SemaphoreType.DMA,
        pltpu.SemaphoreType.DMA,
        pltpu.SemaphoreType.DMA,
        pltpu.VMEM_SHARED((NPAD,), jnp.float32),
        pltpu.VMEM_SHARED((NPAD, HID), jnp.float32),
    ],
)
def _gat2_sc(xl_hbm, xr_hbm, src_hbm, dst_hbm, att_hbm, z1_hbm, z2_hbm,
             out_hbm, w_hbm, src_v, dst_v, xls_v, xrd_v, e_v, a_v, dd_v,
             att_v, ebuf, sem_a, sem_b, sem_c, denom_sh, out_sh):
    c = lax.axis_index("c")
    s = lax.axis_index("s")
    r0 = s * ROWS_T

    pltpu.sync_copy(z1_hbm.at[pl.ds(r0, ROWS_T)],
                    denom_sh.at[pl.ds(r0, ROWS_T)])
    pltpu.sync_copy(z2_hbm.at[pl.ds(r0, ROWS_T)],
                    out_sh.at[pl.ds(r0, ROWS_T)])
    pltpu.sync_copy(att_hbm.at[0], att_v)
    plsc.subcore_barrier()

    att_vecs = [att_v[pl.ds(16 * k, 16)] for k in range(HID // 16)]
    my_blk0 = (c * 16 + s) * BLK_W2
    other_blk0 = ((1 - c) * 16 + s) * BLK_W2

    def passA(blk0, save_w):
        def body(bi, carry):
            base = (blk0 + bi) * EB
            cps = pltpu.async_copy(src_hbm.at[pl.ds(base, EB)], src_v, sem_a)
            cpd = pltpu.async_copy(dst_hbm.at[pl.ds(base, EB)], dst_v, sem_b)
            cps.wait()
            cpl = pltpu.async_copy(xl_hbm.at[src_v], xls_v, sem_a)
            cpd.wait()
            cpr = pltpu.async_copy(xr_hbm.at[dst_v], xrd_v, sem_b)
            cpl.wait()
            cpr.wait()
            _edge_block_logits(xls_v, xrd_v, att_vecs, ebuf, e_v)
            if save_w:
                pltpu.sync_copy(e_v, w_hbm.at[pl.ds(base, EB)])
            pltpu.sync_copy(e_v, denom_sh.at[dst_v], add=True)
            return carry

        lax.fori_loop(0, BLK_W2, body, 0)

    passA(my_blk0, True)
    passA(other_blk0, False)
    plsc.subcore_barrier()

    def passB(bi, carry):
        base = (my_blk0 + bi) * EB
        cps = pltpu.async_copy(src_hbm.at[pl.ds(base, EB)], src_v, sem_a)
        cpd = pltpu.async_copy(dst_hbm.at[pl.ds(base, EB)], dst_v, sem_b)
        cpw = pltpu.async_copy(w_hbm.at[pl.ds(base, EB)], e_v, sem_c)
        cps.wait()
        cpl = pltpu.async_copy(xl_hbm.at[src_v], xls_v, sem_a)
        cpd.wait()
        cpdd = pltpu.async_copy(denom_sh.at[dst_v], dd_v, sem_b)
        cpw.wait()
        cpdd.wait()
        cpl.wait()
        for g in range(EB // 16):
            a_v[pl.ds(16 * g, 16)] = (e_v[pl.ds(16 * g, 16)]
                                      / (dd_v[pl.ds(16 * g, 16)] + 1e-16))
        _scale_rows(xls_v, a_v)
        pltpu.sync_copy(xls_v, out_sh.at[dst_v], add=True)
        return carry

    lax.fori_loop(0, BLK_W2, passB, 0)
    plsc.subcore_barrier()
    pltpu.sync_copy(out_sh.at[pl.ds(r0, ROWS_T)],
                    out_hbm.at[pl.ds(c * NPAD + r0, ROWS_T)])


def kernel(x, edge_index, batch, Wl1, Wr1, att1, b1, Wl2, Wr2, att2, b2,
           Wlin, blin):
    f32 = jnp.float32
    i32 = jnp.int32
    npad_e = EPAD - E_TOT
    loops = jnp.arange(N, dtype=i32)
    pad_src = jnp.arange(npad_e, dtype=i32) % N
    pad_dst = N + jnp.arange(npad_e, dtype=i32) % (NPAD - N)
    src = jnp.concatenate([edge_index[0].astype(i32), loops, pad_src])
    dst = jnp.concatenate([edge_index[1].astype(i32), loops, pad_dst])

    xpad = jnp.pad(x.astype(f32), ((0, NPAD - N), (0, 0)))
    z1 = jnp.zeros((NPAD,), f32)
    z2 = jnp.zeros((NPAD, HID), f32)

    xl1, xr1 = _proj1(xpad, Wl1, Wr1)
    h1, _ = _gat1_sc(xl1, xr1, src, dst, att1, z1, z2)
    xl2, xr2 = _mid(h1.reshape(2, NPAD, HID), b1.reshape(2, HID), Wl2, Wr2)
    h2, _ = _gat2_sc(xl2, xr2, src, dst, att2, z1, z2)

    batch3 = jnp.concatenate(
        [batch.astype(i32), jnp.full((NPAD - N,), NG, i32)]).reshape(NRB, 1, RB)
    wlin_p = jnp.pad(Wlin.astype(f32), ((0, 0), (0, 128 - NC)))
    blin_p = jnp.pad(blin.astype(f32), (0, 128 - NC)).reshape(1, 128)
    logits = _final(h2.reshape(2, NPAD, HID), b2.reshape(1, HID), batch3,
                    wlin_p, blin_p)
    return logits[:, :NC]


# speedup vs baseline: 29.5781x; 1.5697x over previous
import functools

import jax
import jax.numpy as jnp
from jax import lax
from jax.experimental import pallas as pl
from jax.experimental.pallas import tpu as pltpu
from jax.experimental.pallas import tpu_sc as plsc

N = 10000
NPAD = 10240
F_IN = 128
HID = 128
NC = 10
NG = 16
E_TOT = 320000 + N
EB = 96
NBLK = 3456
EPAD = NBLK * EB
BLK_T1 = NBLK // 16
BLK_W2 = NBLK // 32
ROWS_T = NPAD // 16
RB = 1280
NRB = NPAD // RB


def _proj1_body(x_ref, wl_ref, wr_ref, xl_ref, xr_ref):
    x = x_ref[...]
    xl_ref[...] = jnp.dot(x, wl_ref[...], preferred_element_type=jnp.float32)
    xr_ref[...] = jnp.dot(x, wr_ref[...], preferred_element_type=jnp.float32)


def _proj1(xpad, Wl1, Wr1):
    return pl.pallas_call(
        _proj1_body,
        grid=(NRB, 2),
        in_specs=[
            pl.BlockSpec((RB, F_IN), lambda i, h: (i, 0)),
            pl.BlockSpec((F_IN, HID), lambda i, h: (0, h)),
            pl.BlockSpec((F_IN, HID), lambda i, h: (0, h)),
        ],
        out_specs=[
            pl.BlockSpec((RB, HID), lambda i, h: (h * NRB + i, 0)),
            pl.BlockSpec((RB, HID), lambda i, h: (h * NRB + i, 0)),
        ],
        out_shape=[
            jax.ShapeDtypeStruct((2 * NPAD, HID), jnp.float32),
            jax.ShapeDtypeStruct((2 * NPAD, HID), jnp.float32),
        ],
    )(xpad, Wl1, Wr1)


def _elu(v):
    return jnp.where(v > 0, v, jnp.exp(v) - 1.0)


def _mid_body(h_ref, d_ref, b1_ref, wl_ref, wr_ref, xl_ref, xr_ref):
    e0 = _elu(h_ref[0] / (d_ref[0] + 1e-16) + b1_ref[0])
    e1 = _elu(h_ref[1] / (d_ref[1] + 1e-16) + b1_ref[1])
    wl = wl_ref[...]
    wr = wr_ref[...]
    xl_ref[...] = (jnp.dot(e0, wl[:HID], preferred_element_type=jnp.float32)
                   + jnp.dot(e1, wl[HID:], preferred_element_type=jnp.float32))
    xr_ref[...] = (jnp.dot(e0, wr[:HID], preferred_element_type=jnp.float32)
                   + jnp.dot(e1, wr[HID:], preferred_element_type=jnp.float32))


def _mid(h1r, d1r, b1r, Wl2, Wr2):
    return pl.pallas_call(
        _mid_body,
        grid=(NRB,),
        in_specs=[
            pl.BlockSpec((2, RB, HID), lambda i: (0, i, 0)),
            pl.BlockSpec((2, RB, 1), lambda i: (0, i, 0)),
            pl.BlockSpec((2, HID), lambda i: (0, 0)),
            pl.BlockSpec((2 * HID, HID), lambda i: (0, 0)),
            pl.BlockSpec((2 * HID, HID), lambda i: (0, 0)),
        ],
        out_specs=[
            pl.BlockSpec((RB, HID), lambda i: (i, 0)),
            pl.BlockSpec((RB, HID), lambda i: (i, 0)),
        ],
        out_shape=[
            jax.ShapeDtypeStruct((NPAD, HID), jnp.float32),
            jax.ShapeDtypeStruct((NPAD, HID), jnp.float32),
        ],
    )(h1r, d1r, b1r, Wl2, Wr2)


def _final_body(p_ref, d_ref, b2_ref, batch_ref, wlin_ref, blin_ref, out_ref,
                sum_scr, cnt_scr):
    i = pl.program_id(0)

    @pl.when(i == 0)
    def _():
        sum_scr[...] = jnp.zeros((NG, HID), jnp.float32)
        cnt_scr[...] = jnp.zeros((NG, HID), jnp.float32)

    q = (p_ref[0] + p_ref[1]) / (d_ref[0] + d_ref[1] + 1e-16)
    h = _elu(q + b2_ref[0])
    b = batch_ref[0, 0, :]
    P = (lax.broadcasted_iota(jnp.int32, (NG, RB), 0) == b[None, :]
         ).astype(jnp.float32)
    sum_scr[...] += jnp.dot(P, h, preferred_element_type=jnp.float32)
    cnt_scr[...] += jnp.dot(P, jnp.ones((RB, HID), jnp.float32),
                            preferred_element_type=jnp.float32)

    @pl.when(i == NRB - 1)
    def _():
        pooled = sum_scr[...] / jnp.maximum(cnt_scr[...], 1.0)
        out_ref[...] = (jnp.dot(pooled, wlin_ref[...],
                                preferred_element_type=jnp.float32)
                        + blin_ref[0])


def _final(p2r, d2r, b2r, batch3, wlin_p, blin_p):
    return pl.pallas_call(
        _final_body,
        grid=(NRB,),
        in_specs=[
            pl.BlockSpec((2, RB, HID), lambda i: (0, i, 0)),
            pl.BlockSpec((2, RB, 1), lambda i: (0, i, 0)),
            pl.BlockSpec((1, HID), lambda i: (0, 0)),
            pl.BlockSpec((1, 1, RB), lambda i: (i, 0, 0)),
            pl.BlockSpec((HID, 128), lambda i: (0, 0)),
            pl.BlockSpec((1, 128), lambda i: (0, 0)),
        ],
        out_specs=pl.BlockSpec((NG, 128), lambda i: (0, 0)),
        out_shape=jax.ShapeDtypeStruct((NG, 128), jnp.float32),
        scratch_shapes=[
            pltpu.VMEM((NG, HID), jnp.float32),
            pltpu.VMEM((NG, HID), jnp.float32),
        ],
    )(p2r, d2r, b2r, batch3, wlin_p, blin_p)


_MESH = plsc.VectorSubcoreMesh(core_axis_name="c", subcore_axis_name="s")

_SC_SCRATCH = [
    pltpu.VMEM((EB,), jnp.int32),
    pltpu.VMEM((EB,), jnp.int32),
    pltpu.VMEM((EB,), jnp.int32),
    pltpu.VMEM((EB,), jnp.int32),
    pltpu.VMEM((EB, HID), jnp.float32),
    pltpu.VMEM((EB, HID), jnp.float32),
    pltpu.VMEM((EB,), jnp.float32),
    pltpu.VMEM((EB + 16,), jnp.float32),
    pltpu.VMEM((HID,), jnp.float32),
    pltpu.VMEM((EB, 17), jnp.float32),
    pltpu.SemaphoreType.DMA,
    pltpu.SemaphoreType.DMA,
    pltpu.VMEM_SHARED((NPAD,), jnp.float32),
    pltpu.VMEM_SHARED((NPAD, HID), jnp.float32),
]

_SC_OUT = [jax.ShapeDtypeStruct((2 * NPAD, HID), jnp.float32),
           jax.ShapeDtypeStruct((2 * NPAD,), jnp.float32)]


def _edge_block_logits(xls_v, xrd_v, att_vecs, ebuf, e_v):
    @plsc.parallel_loop(0, EB, 1, unroll=4)
    def edge_body(je):
        acc = jnp.zeros((16,), jnp.float32)
        for k in range(HID // 16):
            a = xls_v[je, pl.ds(16 * k, 16)]
            b = xrd_v[je, pl.ds(16 * k, 16)]
            h = a + b
            h = jnp.where(h > 0, h, 0.2 * h)
            acc = acc + h * att_vecs[k]
        ebuf[je, pl.ds(0, 16)] = acc

    iota = lax.broadcasted_iota(jnp.int32, (16,), 0)
    for g in range(EB // 16):
        rows = iota + (16 * g)
        tot = jnp.zeros((16,), jnp.float32)
        for l in range(16):
            tot = tot + plsc.load_gather(
                ebuf, [rows, jnp.full((16,), l, jnp.int32)])
        e_v[pl.ds(16 * g, 16)] = jnp.exp(tot)


def _scale_rows(xls_v, a_v):
    @plsc.parallel_loop(0, EB, 1, unroll=4)
    def body(je):
        a = a_v[pl.ds(je, 16)][0]
        for k in range(HID // 16):
            xls_v[je, pl.ds(16 * k, 16)] = xls_v[je, pl.ds(16 * k, 16)] * a


def _sc_prologue(z1_hbm, z2_hbm, att_hbm, att_row, att_v, denom_sh, out_sh, s):
    r0 = s * ROWS_T
    pltpu.sync_copy(z1_hbm.at[pl.ds(r0, ROWS_T)],
                    denom_sh.at[pl.ds(r0, ROWS_T)])
    pltpu.sync_copy(z2_hbm.at[pl.ds(r0, ROWS_T)],
                    out_sh.at[pl.ds(r0, ROWS_T)])
    pltpu.sync_copy(att_hbm.at[att_row], att_v)
    plsc.subcore_barrier()
    return [att_v[pl.ds(16 * k, 16)] for k in range(HID // 16)]


def _sc_edge_block(base, noff, xl_hbm, xr_hbm, src_hbm, dst_hbm, att_vecs,
                   src_v, dst_v, gidx_v, gidx2_v, xls_v, xrd_v, e_v, a_v,
                   ebuf, sem_a, sem_b, denom_sh, out_sh):
    cps = pltpu.async_copy(src_hbm.at[pl.ds(base, EB)], src_v, sem_a)
    cpd = pltpu.async_copy(dst_hbm.at[pl.ds(base, EB)], dst_v, sem_b)
    cps.wait()
    for g in range(EB // 16):
        gidx_v[pl.ds(16 * g, 16)] = src_v[pl.ds(16 * g, 16)] + noff
    cpl = pltpu.async_copy(xl_hbm.at[gidx_v], xls_v, sem_a)
    cpd.wait()
    for g in range(EB // 16):
        gidx2_v[pl.ds(16 * g, 16)] = dst_v[pl.ds(16 * g, 16)] + noff
    cpr = pltpu.async_copy(xr_hbm.at[gidx2_v], xrd_v, sem_b)
    cpl.wait()
    cpr.wait()
    _edge_block_logits(xls_v, xrd_v, att_vecs, ebuf, e_v)
    for g in range(EB // 16):
        a_v[pl.ds(16 * g, 16)] = e_v[pl.ds(16 * g, 16)]
    _scale_rows(xls_v, a_v)
    cp1 = pltpu.async_copy(e_v, denom_sh.at[dst_v], sem_a, add=True)
    cp2 = pltpu.async_copy(xls_v, out_sh.at[dst_v], sem_b, add=True)
    cp1.wait()
    cp2.wait()


def _sc_epilogue(out_hbm, d_hbm, denom_sh, out_sh, row_off, s):
    plsc.subcore_barrier()
    r0 = s * ROWS_T
    pltpu.sync_copy(out_sh.at[pl.ds(r0, ROWS_T)],
                    out_hbm.at[pl.ds(row_off + r0, ROWS_T)])
    pltpu.sync_copy(denom_sh.at[pl.ds(r0, ROWS_T)],
                    d_hbm.at[pl.ds(row_off + r0, ROWS_T)])


@functools.partial(
    pl.kernel,
    out_type=_SC_OUT,
    mesh=_MESH,
    compiler_params=pltpu.CompilerParams(needs_layout_passes=False),
    scratch_types=_SC_SCRATCH,
)
def _gat1_sc(xl_hbm, xr_hbm, src_hbm, dst_hbm, att_hbm, z1_hbm, z2_hbm,
             out_hbm, d_hbm, src_v, dst_v, gidx_v, gidx2_v, xls_v, xrd_v,
             e_v, a_v, att_v, ebuf, sem_a, sem_b, denom_sh, out_sh):
    c = lax.axis_index("c")
    s = lax.axis_index("s")
    noff = c * NPAD
    att_vecs = _sc_prologue(z1_hbm, z2_hbm, att_hbm, c, att_v, denom_sh,
                            out_sh, s)

    def blk(bi, carry):
        base = (s * BLK_T1 + bi) * EB
        _sc_edge_block(base, noff, xl_hbm, xr_hbm, src_hbm, dst_hbm,
                       att_vecs, src_v, dst_v, gidx_v, gidx2_v, xls_v,
                       xrd_v, e_v, a_v, ebuf, sem_a, sem_b, denom_sh,
                       out_sh)
        return carry

    lax.fori_loop(0, BLK_T1, blk, 0)
    _sc_epilogue(out_hbm, d_hbm, denom_sh, out_sh, noff, s)


@functools.partial(
    pl.kernel,
    out_type=_SC_OUT,
    mesh=_MESH,
    compiler_params=pltpu.CompilerParams(needs_layout_passes=False),
    scratch_types=_SC_SCRATCH,
)
def _gat2_sc(xl_hbm, xr_hbm, src_hbm, dst_hbm, att_hbm, z1_hbm, z2_hbm,
             out_hbm, d_hbm, src_v, dst_v, gidx_v, gidx2_v, xls_v, xrd_v,
             e_v, a_v, att_v, ebuf, sem_a, sem_b, denom_sh, out_sh):
    c = lax.axis_index("c")
    s = lax.axis_index("s")
    att_vecs = _sc_prologue(z1_hbm, z2_hbm, att_hbm, 0, att_v, denom_sh,
                            out_sh, s)
    my_blk0 = (c * 16 + s) * BLK_W2

    def blk(bi, carry):
        base = (my_blk0 + bi) * EB
        _sc_edge_block(base, 0, xl_hbm, xr_hbm, src_hbm, dst_hbm, att_vecs,
                       src_v, dst_v, gidx_v, gidx2_v, xls_v, xrd_v, e_v,
                       a_v, ebuf, sem_a, sem_b, denom_sh, out_sh)
        return carry

    lax.fori_loop(0, BLK_W2, blk, 0)
    _sc_epilogue(out_hbm, d_hbm, denom_sh, out_sh, c * NPAD, s)


def kernel(x, edge_index, batch, Wl1, Wr1, att1, b1, Wl2, Wr2, att2, b2,
           Wlin, blin):
    f32 = jnp.float32
    i32 = jnp.int32
    npad_e = EPAD - E_TOT
    loops = jnp.arange(N, dtype=i32)
    pad_src = jnp.arange(npad_e, dtype=i32) % N
    pad_dst = N + jnp.arange(npad_e, dtype=i32) % (NPAD - N)
    src = jnp.concatenate([edge_index[0].astype(i32), loops, pad_src])
    dst = jnp.concatenate([edge_index[1].astype(i32), loops, pad_dst])

    xpad = jnp.pad(x.astype(f32), ((0, NPAD - N), (0, 0)))
    z1 = jnp.zeros((NPAD,), f32)
    z2 = jnp.zeros((NPAD, HID), f32)

    xl1, xr1 = _proj1(xpad, Wl1, Wr1)
    h1, d1 = _gat1_sc(xl1, xr1, src, dst, att1, z1, z2)
    xl2, xr2 = _mid(h1.reshape(2, NPAD, HID), d1.reshape(2, NPAD, 1),
                    b1.reshape(2, HID), Wl2, Wr2)
    h2, d2 = _gat2_sc(xl2, xr2, src, dst, att2, z1, z2)

    batch3 = jnp.concatenate(
        [batch.astype(i32), jnp.full((NPAD - N,), NG, i32)]).reshape(NRB, 1, RB)
    wlin_p = jnp.pad(Wlin.astype(f32), ((0, 0), (0, 128 - NC)))
    blin_p = jnp.pad(blin.astype(f32), (0, 128 - NC)).reshape(1, 128)
    logits = _final(h2.reshape(2, NPAD, HID), d2.reshape(2, NPAD, 1),
                    b2.reshape(1, HID), batch3, wlin_p, blin_p)
    return logits[:, :NC]


# speedup vs baseline: 36.1534x; 1.2223x over previous
import functools

import jax
import jax.numpy as jnp
from jax import lax
from jax.experimental import pallas as pl
from jax.experimental.pallas import tpu as pltpu
from jax.experimental.pallas import tpu_sc as plsc

N = 10000
NPAD = 10240
F_IN = 128
HID = 128
NC = 10
NG = 16
E_TOT = 320000 + N
EB = 48
NBLK = 6912
EPAD = NBLK * EB
BLK_T1 = NBLK // 16
BLK_W2 = NBLK // 32
ROWS_T = NPAD // 16
RB = 1280
NRB = NPAD // RB


def _proj1_body(x_ref, wl_ref, wr_ref, xl_ref, xr_ref):
    x = x_ref[...]
    xl_ref[...] = jnp.dot(x, wl_ref[...], preferred_element_type=jnp.float32)
    xr_ref[...] = jnp.dot(x, wr_ref[...], preferred_element_type=jnp.float32)


def _proj1(xpad, Wl1, Wr1):
    return pl.pallas_call(
        _proj1_body,
        grid=(NRB, 2),
        in_specs=[
            pl.BlockSpec((RB, F_IN), lambda i, h: (i, 0)),
            pl.BlockSpec((F_IN, HID), lambda i, h: (0, h)),
            pl.BlockSpec((F_IN, HID), lambda i, h: (0, h)),
        ],
        out_specs=[
            pl.BlockSpec((RB, HID), lambda i, h: (h * NRB + i, 0)),
            pl.BlockSpec((RB, HID), lambda i, h: (h * NRB + i, 0)),
        ],
        out_shape=[
            jax.ShapeDtypeStruct((2 * NPAD, HID), jnp.float32),
            jax.ShapeDtypeStruct((2 * NPAD, HID), jnp.float32),
        ],
    )(xpad, Wl1, Wr1)


def _elu(v):
    return jnp.where(v > 0, v, jnp.exp(v) - 1.0)


def _mid_body(h_ref, d_ref, b1_ref, wl_ref, wr_ref, xl_ref, xr_ref):
    e0 = _elu(h_ref[0] / (d_ref[0] + 1e-16) + b1_ref[0])
    e1 = _elu(h_ref[1] / (d_ref[1] + 1e-16) + b1_ref[1])
    wl = wl_ref[...]
    wr = wr_ref[...]
    xl_ref[...] = (jnp.dot(e0, wl[:HID], preferred_element_type=jnp.float32)
                   + jnp.dot(e1, wl[HID:], preferred_element_type=jnp.float32))
    xr_ref[...] = (jnp.dot(e0, wr[:HID], preferred_element_type=jnp.float32)
                   + jnp.dot(e1, wr[HID:], preferred_element_type=jnp.float32))


def _mid(h1r, d1r, b1r, Wl2, Wr2):
    return pl.pallas_call(
        _mid_body,
        grid=(NRB,),
        in_specs=[
            pl.BlockSpec((2, RB, HID), lambda i: (0, i, 0)),
            pl.BlockSpec((2, RB, 1), lambda i: (0, i, 0)),
            pl.BlockSpec((2, HID), lambda i: (0, 0)),
            pl.BlockSpec((2 * HID, HID), lambda i: (0, 0)),
            pl.BlockSpec((2 * HID, HID), lambda i: (0, 0)),
        ],
        out_specs=[
            pl.BlockSpec((RB, HID), lambda i: (i, 0)),
            pl.BlockSpec((RB, HID), lambda i: (i, 0)),
        ],
        out_shape=[
            jax.ShapeDtypeStruct((NPAD, HID), jnp.float32),
            jax.ShapeDtypeStruct((NPAD, HID), jnp.float32),
        ],
    )(h1r, d1r, b1r, Wl2, Wr2)


def _final_body(p_ref, d_ref, b2_ref, batch_ref, wlin_ref, blin_ref, out_ref,
                sum_scr, cnt_scr):
    i = pl.program_id(0)

    @pl.when(i == 0)
    def _():
        sum_scr[...] = jnp.zeros((NG, HID), jnp.float32)
        cnt_scr[...] = jnp.zeros((NG, HID), jnp.float32)

    q = (p_ref[0] + p_ref[1]) / (d_ref[0] + d_ref[1] + 1e-16)
    h = _elu(q + b2_ref[0])
    b = batch_ref[0, 0, :]
    P = (lax.broadcasted_iota(jnp.int32, (NG, RB), 0) == b[None, :]
         ).astype(jnp.float32)
    sum_scr[...] += jnp.dot(P, h, preferred_element_type=jnp.float32)
    cnt_scr[...] += jnp.dot(P, jnp.ones((RB, HID), jnp.float32),
                            preferred_element_type=jnp.float32)

    @pl.when(i == NRB - 1)
    def _():
        pooled = sum_scr[...] / jnp.maximum(cnt_scr[...], 1.0)
        out_ref[...] = (jnp.dot(pooled, wlin_ref[...],
                                preferred_element_type=jnp.float32)
                        + blin_ref[0])


def _final(p2r, d2r, b2r, batch3, wlin_p, blin_p):
    return pl.pallas_call(
        _final_body,
        grid=(NRB,),
        in_specs=[
            pl.BlockSpec((2, RB, HID), lambda i: (0, i, 0)),
            pl.BlockSpec((2, RB, 1), lambda i: (0, i, 0)),
            pl.BlockSpec((1, HID), lambda i: (0, 0)),
            pl.BlockSpec((1, 1, RB), lambda i: (i, 0, 0)),
            pl.BlockSpec((HID, 128), lambda i: (0, 0)),
            pl.BlockSpec((1, 128), lambda i: (0, 0)),
        ],
        out_specs=pl.BlockSpec((NG, 128), lambda i: (0, 0)),
        out_shape=jax.ShapeDtypeStruct((NG, 128), jnp.float32),
        scratch_shapes=[
            pltpu.VMEM((NG, HID), jnp.float32),
            pltpu.VMEM((NG, HID), jnp.float32),
        ],
    )(p2r, d2r, b2r, batch3, wlin_p, blin_p)


_MESH = plsc.VectorSubcoreMesh(core_axis_name="c", subcore_axis_name="s")

_BUFSET = [
    pltpu.VMEM((EB,), jnp.int32),
    pltpu.VMEM((EB,), jnp.int32),
    pltpu.VMEM((EB,), jnp.int32),
    pltpu.VMEM((EB,), jnp.int32),
    pltpu.VMEM((EB, HID), jnp.float32),
    pltpu.VMEM((EB, HID), jnp.float32),
    pltpu.VMEM((EB,), jnp.float32),
    pltpu.VMEM((EB + 16,), jnp.float32),
]
_SC_SCRATCH = (_BUFSET + _BUFSET + [
    pltpu.VMEM((HID,), jnp.float32),
    pltpu.VMEM((EB, 17), jnp.float32),
] + [pltpu.SemaphoreType.DMA] * 10
  + [
    pltpu.VMEM_SHARED((NPAD,), jnp.float32),
    pltpu.VMEM_SHARED((NPAD, HID), jnp.float32),
])

_SC_OUT = [jax.ShapeDtypeStruct((2 * NPAD, HID), jnp.float32),
           jax.ShapeDtypeStruct((2 * NPAD,), jnp.float32)]


def _edge_block_logits(xls_v, xrd_v, att_vecs, ebuf, e_v):
    @plsc.parallel_loop(0, EB, 1, unroll=4)
    def edge_body(je):
        acc = jnp.zeros((16,), jnp.float32)
        for k in range(HID // 16):
            a = xls_v[je, pl.ds(16 * k, 16)]
            b = xrd_v[je, pl.ds(16 * k, 16)]
            h = a + b
            h = jnp.where(h > 0, h, 0.2 * h)
            acc = acc + h * att_vecs[k]
        ebuf[je, pl.ds(0, 16)] = acc

    iota = lax.broadcasted_iota(jnp.int32, (16,), 0)
    for g in range(EB // 16):
        rows = iota + (16 * g)
        tot = jnp.zeros((16,), jnp.float32)
        for l in range(16):
            tot = tot + plsc.load_gather(
                ebuf, [rows, jnp.full((16,), l, jnp.int32)])
        e_v[pl.ds(16 * g, 16)] = jnp.exp(tot)


def _scale_rows(xls_v, a_v):
    @plsc.parallel_loop(0, EB, 1, unroll=4)
    def body(je):
        a = a_v[pl.ds(je, 16)][0]
        for k in range(HID // 16):
            xls_v[je, pl.ds(16 * k, 16)] = xls_v[je, pl.ds(16 * k, 16)] * a


def _sc_prologue(z1_hbm, z2_hbm, att_hbm, att_row, att_v, denom_sh, out_sh, s):
    r0 = s * ROWS_T
    pltpu.sync_copy(z1_hbm.at[pl.ds(r0, ROWS_T)],
                    denom_sh.at[pl.ds(r0, ROWS_T)])
    pltpu.sync_copy(z2_hbm.at[pl.ds(r0, ROWS_T)],
                    out_sh.at[pl.ds(r0, ROWS_T)])
    pltpu.sync_copy(att_hbm.at[att_row], att_v)
    plsc.subcore_barrier()
    return [att_v[pl.ds(16 * k, 16)] for k in range(HID // 16)]


def _sc_pipeline(nblk_t, base_of, noff, xl_hbm, xr_hbm, src_hbm, dst_hbm,
                 att_vecs, bufs, isems, gsems, sem_sa, sem_sb, ebuf,
                 denom_sh, out_sh):
    def issue_idx(bi, p):
        base = base_of(bi)
        pltpu.async_copy(src_hbm.at[pl.ds(base, EB)], bufs[p][0],
                         isems[p][0])
        pltpu.async_copy(dst_hbm.at[pl.ds(base, EB)], bufs[p][1],
                         isems[p][1])

    def wait_idx(p):
        pltpu.make_async_copy(src_hbm.at[pl.ds(0, EB)], bufs[p][0],
                              isems[p][0]).wait()
        pltpu.make_async_copy(dst_hbm.at[pl.ds(0, EB)], bufs[p][1],
                              isems[p][1]).wait()

    def issue_gathers(p):
        src_v, dst_v, gidx_v, gidx2_v, xls_v, xrd_v = bufs[p][:6]
        for g in range(EB // 16):
            gidx_v[pl.ds(16 * g, 16)] = src_v[pl.ds(16 * g, 16)] + noff
            gidx2_v[pl.ds(16 * g, 16)] = dst_v[pl.ds(16 * g, 16)] + noff
        pltpu.async_copy(xl_hbm.at[gidx_v], xls_v, gsems[p][0])
        pltpu.async_copy(xr_hbm.at[gidx2_v], xrd_v, gsems[p][1])

    def wait_gathers(p):
        pltpu.make_async_copy(xl_hbm.at[pl.ds(0, EB)], bufs[p][4],
                              gsems[p][0]).wait()
        pltpu.make_async_copy(xl_hbm.at[pl.ds(0, EB)], bufs[p][5],
                              gsems[p][1]).wait()

    issue_idx(0, 0)
    wait_idx(0)
    issue_gathers(0)
    issue_idx(1, 1)

    def pair(j, carry):
        for p in (0, 1):
            q = 1 - p
            i = 2 * j + p
            wait_idx(q)
            issue_gathers(q)
            wait_gathers(p)
            dst_v = bufs[p][1]
            xls_v = bufs[p][4]
            e_v = bufs[p][6]
            a_v = bufs[p][7]
            _edge_block_logits(xls_v, bufs[p][5], att_vecs, ebuf, e_v)
            for g in range(EB // 16):
                a_v[pl.ds(16 * g, 16)] = e_v[pl.ds(16 * g, 16)]
            _scale_rows(xls_v, a_v)
            cp1 = pltpu.async_copy(e_v, denom_sh.at[dst_v], sem_sa,
                                   add=True)
            cp2 = pltpu.async_copy(xls_v, out_sh.at[dst_v], sem_sb,
                                   add=True)
            cp1.wait()
            cp2.wait()
            issue_idx(lax.rem(i + 2, nblk_t), p)
        return carry

    lax.fori_loop(0, nblk_t // 2, pair, 0)
    wait_gathers(0)
    wait_idx(1)


def _sc_epilogue(out_hbm, d_hbm, denom_sh, out_sh, row_off, s):
    plsc.subcore_barrier()
    r0 = s * ROWS_T
    pltpu.sync_copy(out_sh.at[pl.ds(r0, ROWS_T)],
                    out_hbm.at[pl.ds(row_off + r0, ROWS_T)])
    pltpu.sync_copy(denom_sh.at[pl.ds(r0, ROWS_T)],
                    d_hbm.at[pl.ds(row_off + r0, ROWS_T)])


@functools.partial(
    pl.kernel,
    out_type=_SC_OUT,
    mesh=_MESH,
    compiler_params=pltpu.CompilerParams(needs_layout_passes=False),
    scratch_types=_SC_SCRATCH,
)
def _gat1_sc(xl_hbm, xr_hbm, src_hbm, dst_hbm, att_hbm, z1_hbm, z2_hbm,
             out_hbm, d_hbm,
             b0_src, b0_dst, b0_gi, b0_gi2, b0_xls, b0_xrd, b0_e, b0_a,
             b1_src, b1_dst, b1_gi, b1_gi2, b1_xls, b1_xrd, b1_e, b1_a,
             att_v, ebuf, i_a0, i_b0, i_a1, i_b1, g_a0, g_b0, g_a1, g_b1,
             s_a, s_b, denom_sh, out_sh):
    c = lax.axis_index("c")
    s = lax.axis_index("s")
    noff = c * NPAD
    att_vecs = _sc_prologue(z1_hbm, z2_hbm, att_hbm, c, att_v, denom_sh,
                            out_sh, s)
    bufs = ((b0_src, b0_dst, b0_gi, b0_gi2, b0_xls, b0_xrd, b0_e, b0_a),
            (b1_src, b1_dst, b1_gi, b1_gi2, b1_xls, b1_xrd, b1_e, b1_a))
    blk0 = s * BLK_T1
    _sc_pipeline(BLK_T1, lambda bi: (blk0 + bi) * EB, noff, xl_hbm, xr_hbm,
                 src_hbm, dst_hbm, att_vecs, bufs,
                 ((i_a0, i_b0), (i_a1, i_b1)), ((g_a0, g_b0), (g_a1, g_b1)),
                 s_a, s_b, ebuf, denom_sh, out_sh)
    _sc_epilogue(out_hbm, d_hbm, denom_sh, out_sh, noff, s)


@functools.partial(
    pl.kernel,
    out_type=_SC_OUT,
    mesh=_MESH,
    compiler_params=pltpu.CompilerParams(needs_layout_passes=False),
    scratch_types=_SC_SCRATCH,
)
def _gat2_sc(xl_hbm, xr_hbm, src_hbm, dst_hbm, att_hbm, z1_hbm, z2_hbm,
             out_hbm, d_hbm,
             b0_src, b0_dst, b0_gi, b0_gi2, b0_xls, b0_xrd, b0_e, b0_a,
             b1_src, b1_dst, b1_gi, b1_gi2, b1_xls, b1_xrd, b1_e, b1_a,
             att_v, ebuf, i_a0, i_b0, i_a1, i_b1, g_a0, g_b0, g_a1, g_b1,
             s_a, s_b, denom_sh, out_sh):
    c = lax.axis_index("c")
    s = lax.axis_index("s")
    att_vecs = _sc_prologue(z1_hbm, z2_hbm, att_hbm, 0, att_v, denom_sh,
                            out_sh, s)
    bufs = ((b0_src, b0_dst, b0_gi, b0_gi2, b0_xls, b0_xrd, b0_e, b0_a),
            (b1_src, b1_dst, b1_gi, b1_gi2, b1_xls, b1_xrd, b1_e, b1_a))
    my_blk0 = (c * 16 + s) * BLK_W2
    _sc_pipeline(BLK_W2, lambda bi: (my_blk0 + bi) * EB, 0, xl_hbm, xr_hbm,
                 src_hbm, dst_hbm, att_vecs, bufs,
                 ((i_a0, i_b0), (i_a1, i_b1)), ((g_a0, g_b0), (g_a1, g_b1)),
                 s_a, s_b, ebuf, denom_sh, out_sh)
    _sc_epilogue(out_hbm, d_hbm, denom_sh, out_sh, c * NPAD, s)


def kernel(x, edge_index, batch, Wl1, Wr1, att1, b1, Wl2, Wr2, att2, b2,
           Wlin, blin):
    f32 = jnp.float32
    i32 = jnp.int32
    npad_e = EPAD - E_TOT
    loops = jnp.arange(N, dtype=i32)
    pad_src = jnp.arange(npad_e, dtype=i32) % N
    pad_dst = N + jnp.arange(npad_e, dtype=i32) % (NPAD - N)
    src = jnp.concatenate([edge_index[0].astype(i32), loops, pad_src])
    dst = jnp.concatenate([edge_index[1].astype(i32), loops, pad_dst])

    xpad = jnp.pad(x.astype(f32), ((0, NPAD - N), (0, 0)))
    z1 = jnp.zeros((NPAD,), f32)
    z2 = jnp.zeros((NPAD, HID), f32)

    xl1, xr1 = _proj1(xpad, Wl1, Wr1)
    h1, d1 = _gat1_sc(xl1, xr1, src, dst, att1, z1, z2)
    xl2, xr2 = _mid(h1.reshape(2, NPAD, HID), d1.reshape(2, NPAD, 1),
                    b1.reshape(2, HID), Wl2, Wr2)
    h2, d2 = _gat2_sc(xl2, xr2, src, dst, att2, z1, z2)

    batch3 = jnp.concatenate(
        [batch.astype(i32), jnp.full((NPAD - N,), NG, i32)]).reshape(NRB, 1, RB)
    wlin_p = jnp.pad(Wlin.astype(f32), ((0, 0), (0, 128 - NC)))
    blin_p = jnp.pad(blin.astype(f32), (0, 128 - NC)).reshape(1, 128)
    logits = _final(h2.reshape(2, NPAD, HID), d2.reshape(2, NPAD, 1),
                    b2.reshape(1, HID), batch3, wlin_p, blin_p)
    return logits[:, :NC]


# speedup vs baseline: 41.9309x; 1.1598x over previous
import functools

import jax
import jax.numpy as jnp
from jax import lax
from jax.experimental import pallas as pl
from jax.experimental.pallas import tpu as pltpu
from jax.experimental.pallas import tpu_sc as plsc

N = 10000
NPAD = 10240
F_IN = 128
HID = 128
NC = 10
NG = 16
E_TOT = 320000 + N
EB = 48
NBLK = 6912
EPAD = NBLK * EB
BLK_T1 = NBLK // 16
BLK_W2 = NBLK // 32
ROWS_T = NPAD // 16
RB = 1280
NRB = NPAD // RB


def _proj1_body(x_ref, wl_ref, wr_ref, xl_ref, xr_ref):
    x = x_ref[...]
    xl_ref[...] = jnp.dot(x, wl_ref[...], preferred_element_type=jnp.float32)
    xr_ref[...] = jnp.dot(x, wr_ref[...], preferred_element_type=jnp.float32)


def _proj1(xpad, Wl1, Wr1):
    return pl.pallas_call(
        _proj1_body,
        grid=(NRB, 2),
        in_specs=[
            pl.BlockSpec((RB, F_IN), lambda i, h: (i, 0)),
            pl.BlockSpec((F_IN, HID), lambda i, h: (0, h)),
            pl.BlockSpec((F_IN, HID), lambda i, h: (0, h)),
        ],
        out_specs=[
            pl.BlockSpec((RB, HID), lambda i, h: (h * NRB + i, 0)),
            pl.BlockSpec((RB, HID), lambda i, h: (h * NRB + i, 0)),
        ],
        out_shape=[
            jax.ShapeDtypeStruct((2 * NPAD, HID), jnp.float32),
            jax.ShapeDtypeStruct((2 * NPAD, HID), jnp.float32),
        ],
    )(xpad, Wl1, Wr1)


def _elu(v):
    return jnp.where(v > 0, v, jnp.exp(v) - 1.0)


def _mid_body(h_ref, d_ref, b1_ref, wl_ref, wr_ref, xl_ref, xr_ref):
    e0 = _elu(h_ref[0] / (d_ref[0] + 1e-16) + b1_ref[0])
    e1 = _elu(h_ref[1] / (d_ref[1] + 1e-16) + b1_ref[1])
    wl = wl_ref[...]
    wr = wr_ref[...]
    xl_ref[...] = (jnp.dot(e0, wl[:HID], preferred_element_type=jnp.float32)
                   + jnp.dot(e1, wl[HID:], preferred_element_type=jnp.float32))
    xr_ref[...] = (jnp.dot(e0, wr[:HID], preferred_element_type=jnp.float32)
                   + jnp.dot(e1, wr[HID:], preferred_element_type=jnp.float32))


def _mid(h1r, d1r, b1r, Wl2, Wr2):
    return pl.pallas_call(
        _mid_body,
        grid=(NRB,),
        in_specs=[
            pl.BlockSpec((2, RB, HID), lambda i: (0, i, 0)),
            pl.BlockSpec((2, RB, 1), lambda i: (0, i, 0)),
            pl.BlockSpec((2, HID), lambda i: (0, 0)),
            pl.BlockSpec((2 * HID, HID), lambda i: (0, 0)),
            pl.BlockSpec((2 * HID, HID), lambda i: (0, 0)),
        ],
        out_specs=[
            pl.BlockSpec((RB, HID), lambda i: (i, 0)),
            pl.BlockSpec((RB, HID), lambda i: (i, 0)),
        ],
        out_shape=[
            jax.ShapeDtypeStruct((NPAD, HID), jnp.float32),
            jax.ShapeDtypeStruct((NPAD, HID), jnp.float32),
        ],
    )(h1r, d1r, b1r, Wl2, Wr2)


def _final_body(p_ref, d_ref, b2_ref, batch_ref, wlin_ref, blin_ref, out_ref,
                sum_scr, cnt_scr):
    i = pl.program_id(0)

    @pl.when(i == 0)
    def _():
        sum_scr[...] = jnp.zeros((NG, HID), jnp.float32)
        cnt_scr[...] = jnp.zeros((NG, HID), jnp.float32)

    q = (p_ref[0] + p_ref[1]) / (d_ref[0] + d_ref[1] + 1e-16)
    h = _elu(q + b2_ref[0])
    b = batch_ref[0, 0, :]
    P = (lax.broadcasted_iota(jnp.int32, (NG, RB), 0) == b[None, :]
         ).astype(jnp.float32)
    sum_scr[...] += jnp.dot(P, h, preferred_element_type=jnp.float32)
    cnt_scr[...] += jnp.dot(P, jnp.ones((RB, HID), jnp.float32),
                            preferred_element_type=jnp.float32)

    @pl.when(i == NRB - 1)
    def _():
        pooled = sum_scr[...] / jnp.maximum(cnt_scr[...], 1.0)
        out_ref[...] = (jnp.dot(pooled, wlin_ref[...],
                                preferred_element_type=jnp.float32)
                        + blin_ref[0])


def _final(p2r, d2r, b2r, batch3, wlin_p, blin_p):
    return pl.pallas_call(
        _final_body,
        grid=(NRB,),
        in_specs=[
            pl.BlockSpec((2, RB, HID), lambda i: (0, i, 0)),
            pl.BlockSpec((2, RB, 1), lambda i: (0, i, 0)),
            pl.BlockSpec((1, HID), lambda i: (0, 0)),
            pl.BlockSpec((1, 1, RB), lambda i: (i, 0, 0)),
            pl.BlockSpec((HID, 128), lambda i: (0, 0)),
            pl.BlockSpec((1, 128), lambda i: (0, 0)),
        ],
        out_specs=pl.BlockSpec((NG, 128), lambda i: (0, 0)),
        out_shape=jax.ShapeDtypeStruct((NG, 128), jnp.float32),
        scratch_shapes=[
            pltpu.VMEM((NG, HID), jnp.float32),
            pltpu.VMEM((NG, HID), jnp.float32),
        ],
    )(p2r, d2r, b2r, batch3, wlin_p, blin_p)


_MESH = plsc.VectorSubcoreMesh(core_axis_name="c", subcore_axis_name="s")

_BUFSET = [
    pltpu.VMEM((EB,), jnp.int32),
    pltpu.VMEM((EB,), jnp.int32),
    pltpu.VMEM((EB,), jnp.int32),
    pltpu.VMEM((EB,), jnp.int32),
    pltpu.VMEM((EB, HID), jnp.float32),
    pltpu.VMEM((EB, HID), jnp.float32),
    pltpu.VMEM((EB,), jnp.float32),
    pltpu.VMEM((EB + 16,), jnp.float32),
    pltpu.VMEM((EB,), jnp.int32),
]
_SC_SCRATCH = (_BUFSET + _BUFSET + [
    pltpu.VMEM((HID,), jnp.float32),
    pltpu.VMEM((EB, 17), jnp.float32),
] + [pltpu.SemaphoreType.DMA] * 12
  + [
    pltpu.VMEM_SHARED((NPAD,), jnp.float32),
    pltpu.VMEM_SHARED((NPAD, HID), jnp.float32),
])

_SC_OUT = [jax.ShapeDtypeStruct((2 * NPAD, HID), jnp.float32),
           jax.ShapeDtypeStruct((2 * NPAD,), jnp.float32)]


def _edge_block_logits(xls_v, xrd_v, att_vecs, ebuf, e_v):
    @plsc.parallel_loop(0, EB, 1, unroll=6)
    def edge_body(je):
        acc = jnp.zeros((16,), jnp.float32)
        for k in range(HID // 16):
            a = xls_v[je, pl.ds(16 * k, 16)]
            b = xrd_v[je, pl.ds(16 * k, 16)]
            h = a + b
            h = jnp.where(h > 0, h, 0.2 * h)
            acc = acc + h * att_vecs[k]
        ebuf[je, pl.ds(0, 16)] = acc

    iota = lax.broadcasted_iota(jnp.int32, (16,), 0)
    for g in range(EB // 16):
        rows = iota + (16 * g)
        tot = jnp.zeros((16,), jnp.float32)
        for l in range(16):
            tot = tot + plsc.load_gather(
                ebuf, [rows, jnp.full((16,), l, jnp.int32)])
        e_v[pl.ds(16 * g, 16)] = jnp.exp(tot)


def _scale_rows(xls_v, a_v):
    @plsc.parallel_loop(0, EB, 1, unroll=6)
    def body(je):
        a = a_v[pl.ds(je, 16)][0]
        for k in range(HID // 16):
            xls_v[je, pl.ds(16 * k, 16)] = xls_v[je, pl.ds(16 * k, 16)] * a


def _sc_prologue(z1_hbm, z2_hbm, att_hbm, att_row, att_v, denom_sh, out_sh, s):
    r0 = s * ROWS_T
    pltpu.sync_copy(z1_hbm.at[pl.ds(r0, ROWS_T)],
                    denom_sh.at[pl.ds(r0, ROWS_T)])
    pltpu.sync_copy(z2_hbm.at[pl.ds(r0, ROWS_T)],
                    out_sh.at[pl.ds(r0, ROWS_T)])
    pltpu.sync_copy(att_hbm.at[att_row], att_v)
    plsc.subcore_barrier()
    return [att_v[pl.ds(16 * k, 16)] for k in range(HID // 16)]


def _sc_pipeline(nblk_t, base_of, noff, with_off, xl_hbm, xr_hbm, src_hbm,
                 dst_hbm, att_vecs, bufs, isems, gsems, ssems, ebuf,
                 denom_sh, out_sh):
    def issue_idx(bi, p):
        base = base_of(bi)
        pltpu.async_copy(src_hbm.at[pl.ds(base, EB)], bufs[p][0],
                         isems[p][0])
        pltpu.async_copy(dst_hbm.at[pl.ds(base, EB)], bufs[p][1],
                         isems[p][1])

    def wait_idx(p):
        pltpu.make_async_copy(src_hbm.at[pl.ds(0, EB)], bufs[p][0],
                              isems[p][0]).wait()
        pltpu.make_async_copy(dst_hbm.at[pl.ds(0, EB)], bufs[p][1],
                              isems[p][1]).wait()

    def issue_gathers(p):
        src_v, dst_v, gidx_v, gidx2_v, xls_v, xrd_v = bufs[p][:6]
        if with_off:
            for g in range(EB // 16):
                gidx_v[pl.ds(16 * g, 16)] = src_v[pl.ds(16 * g, 16)] + noff
                gidx2_v[pl.ds(16 * g, 16)] = dst_v[pl.ds(16 * g, 16)] + noff
            pltpu.async_copy(xl_hbm.at[gidx_v], xls_v, gsems[p][0])
            pltpu.async_copy(xr_hbm.at[gidx2_v], xrd_v, gsems[p][1])
        else:
            pltpu.async_copy(xl_hbm.at[src_v], xls_v, gsems[p][0])
            pltpu.async_copy(xr_hbm.at[dst_v], xrd_v, gsems[p][1])

    def wait_gathers(p):
        pltpu.make_async_copy(xl_hbm.at[pl.ds(0, EB)], bufs[p][4],
                              gsems[p][0]).wait()
        pltpu.make_async_copy(xl_hbm.at[pl.ds(0, EB)], bufs[p][5],
                              gsems[p][1]).wait()

    def wait_scatters(p):
        pltpu.make_async_copy(bufs[p][6], denom_sh.at[pl.ds(0, EB)],
                              ssems[p][0]).wait()
        pltpu.make_async_copy(bufs[p][4], out_sh.at[pl.ds(0, EB)],
                              ssems[p][1]).wait()

    issue_idx(0, 0)
    wait_idx(0)
    issue_gathers(0)
    issue_idx(1, 1)

    def pair(j, carry):
        for p in (0, 1):
            q = 1 - p
            i = 2 * j + p
            wait_idx(q)
            if p == 0:
                @pl.when(j > 0)
                def _():
                    wait_scatters(q)
            else:
                wait_scatters(q)
            issue_gathers(q)
            wait_gathers(p)
            dst_v = bufs[p][1]
            xls_v = bufs[p][4]
            e_v = bufs[p][6]
            a_v = bufs[p][7]
            dscat_v = bufs[p][8]
            _edge_block_logits(xls_v, bufs[p][5], att_vecs, ebuf, e_v)
            for g in range(EB // 16):
                a_v[pl.ds(16 * g, 16)] = e_v[pl.ds(16 * g, 16)]
                dscat_v[pl.ds(16 * g, 16)] = dst_v[pl.ds(16 * g, 16)]
            _scale_rows(xls_v, a_v)
            pltpu.async_copy(e_v, denom_sh.at[dscat_v], ssems[p][0],
                             add=True)
            pltpu.async_copy(xls_v, out_sh.at[dscat_v], ssems[p][1],
                             add=True)
            issue_idx(lax.rem(i + 2, nblk_t), p)
        return carry

    lax.fori_loop(0, nblk_t // 2, pair, 0)
    wait_scatters(1)
    wait_gathers(0)
    wait_idx(1)


def _sc_epilogue(out_hbm, d_hbm, denom_sh, out_sh, row_off, s):
    plsc.subcore_barrier()
    r0 = s * ROWS_T
    pltpu.sync_copy(out_sh.at[pl.ds(r0, ROWS_T)],
                    out_hbm.at[pl.ds(row_off + r0, ROWS_T)])
    pltpu.sync_copy(denom_sh.at[pl.ds(r0, ROWS_T)],
                    d_hbm.at[pl.ds(row_off + r0, ROWS_T)])


@functools.partial(
    pl.kernel,
    out_type=_SC_OUT,
    mesh=_MESH,
    compiler_params=pltpu.CompilerParams(needs_layout_passes=False),
    scratch_types=_SC_SCRATCH,
)
def _gat1_sc(xl_hbm, xr_hbm, src_hbm, dst_hbm, att_hbm, z1_hbm, z2_hbm,
             out_hbm, d_hbm,
             b0_src, b0_dst, b0_gi, b0_gi2, b0_xls, b0_xrd, b0_e, b0_a,
             b0_ds, b1_src, b1_dst, b1_gi, b1_gi2, b1_xls, b1_xrd, b1_e,
             b1_a, b1_ds, att_v, ebuf, i_a0, i_b0, i_a1, i_b1, g_a0, g_b0,
             g_a1, g_b1, s_a0, s_b0, s_a1, s_b1, denom_sh, out_sh):
    c = lax.axis_index("c")
    s = lax.axis_index("s")
    noff = c * NPAD
    att_vecs = _sc_prologue(z1_hbm, z2_hbm, att_hbm, c, att_v, denom_sh,
                            out_sh, s)
    bufs = ((b0_src, b0_dst, b0_gi, b0_gi2, b0_xls, b0_xrd, b0_e, b0_a,
             b0_ds),
            (b1_src, b1_dst, b1_gi, b1_gi2, b1_xls, b1_xrd, b1_e, b1_a,
             b1_ds))
    blk0 = s * BLK_T1
    _sc_pipeline(BLK_T1, lambda bi: (blk0 + bi) * EB, noff, True, xl_hbm,
                 xr_hbm, src_hbm, dst_hbm, att_vecs, bufs,
                 ((i_a0, i_b0), (i_a1, i_b1)), ((g_a0, g_b0), (g_a1, g_b1)),
                 ((s_a0, s_b0), (s_a1, s_b1)), ebuf, denom_sh, out_sh)
    _sc_epilogue(out_hbm, d_hbm, denom_sh, out_sh, noff, s)


@functools.partial(
    pl.kernel,
    out_type=_SC_OUT,
    mesh=_MESH,
    compiler_params=pltpu.CompilerParams(needs_layout_passes=False),
    scratch_types=_SC_SCRATCH,
)
def _gat2_sc(xl_hbm, xr_hbm, src_hbm, dst_hbm, att_hbm, z1_hbm, z2_hbm,
             out_hbm, d_hbm,
             b0_src, b0_dst, b0_gi, b0_gi2, b0_xls, b0_xrd, b0_e, b0_a,
             b0_ds, b1_src, b1_dst, b1_gi, b1_gi2, b1_xls, b1_xrd, b1_e,
             b1_a, b1_ds, att_v, ebuf, i_a0, i_b0, i_a1, i_b1, g_a0, g_b0,
             g_a1, g_b1, s_a0, s_b0, s_a1, s_b1, denom_sh, out_sh):
    c = lax.axis_index("c")
    s = lax.axis_index("s")
    att_vecs = _sc_prologue(z1_hbm, z2_hbm, att_hbm, 0, att_v, denom_sh,
                            out_sh, s)
    bufs = ((b0_src, b0_dst, b0_gi, b0_gi2, b0_xls, b0_xrd, b0_e, b0_a,
             b0_ds),
            (b1_src, b1_dst, b1_gi, b1_gi2, b1_xls, b1_xrd, b1_e, b1_a,
             b1_ds))
    my_blk0 = (c * 16 + s) * BLK_W2
    _sc_pipeline(BLK_W2, lambda bi: (my_blk0 + bi) * EB, 0, False, xl_hbm,
                 xr_hbm, src_hbm, dst_hbm, att_vecs, bufs,
                 ((i_a0, i_b0), (i_a1, i_b1)), ((g_a0, g_b0), (g_a1, g_b1)),
                 ((s_a0, s_b0), (s_a1, s_b1)), ebuf, denom_sh, out_sh)
    _sc_epilogue(out_hbm, d_hbm, denom_sh, out_sh, c * NPAD, s)


def kernel(x, edge_index, batch, Wl1, Wr1, att1, b1, Wl2, Wr2, att2, b2,
           Wlin, blin):
    f32 = jnp.float32
    i32 = jnp.int32
    npad_e = EPAD - E_TOT
    loops = jnp.arange(N, dtype=i32)
    pad_src = jnp.arange(npad_e, dtype=i32) % N
    pad_dst = N + jnp.arange(npad_e, dtype=i32) % (NPAD - N)
    src = jnp.concatenate([edge_index[0].astype(i32), loops, pad_src])
    dst = jnp.concatenate([edge_index[1].astype(i32), loops, pad_dst])

    xpad = jnp.pad(x.astype(f32), ((0, NPAD - N), (0, 0)))
    z1 = jnp.zeros((NPAD,), f32)
    z2 = jnp.zeros((NPAD, HID), f32)

    xl1, xr1 = _proj1(xpad, Wl1, Wr1)
    h1, d1 = _gat1_sc(xl1, xr1, src, dst, att1, z1, z2)
    xl2, xr2 = _mid(h1.reshape(2, NPAD, HID), d1.reshape(2, NPAD, 1),
                    b1.reshape(2, HID), Wl2, Wr2)
    h2, d2 = _gat2_sc(xl2, xr2, src, dst, att2, z1, z2)

    batch3 = jnp.concatenate(
        [batch.astype(i32), jnp.full((NPAD - N,), NG, i32)]).reshape(NRB, 1, RB)
    wlin_p = jnp.pad(Wlin.astype(f32), ((0, 0), (0, 128 - NC)))
    blin_p = jnp.pad(blin.astype(f32), (0, 128 - NC)).reshape(1, 128)
    logits = _final(h2.reshape(2, NPAD, HID), d2.reshape(2, NPAD, 1),
                    b2.reshape(1, HID), batch3, wlin_p, blin_p)
    return logits[:, :NC]


# speedup vs baseline: 43.9275x; 1.0476x over previous
import functools

import jax
import jax.numpy as jnp
from jax import lax
from jax.experimental import pallas as pl
from jax.experimental.pallas import tpu as pltpu
from jax.experimental.pallas import tpu_sc as plsc

N = 10000
NPAD = 10240
F_IN = 128
HID = 128
NC = 10
NG = 16
E_TOT = 320000 + N
EB = 64
NBLK = 5184
EPAD = NBLK * EB
BLK_T1 = NBLK // 16
BLK_W2 = NBLK // 32
ROWS_T = NPAD // 16
RB = 1280
NRB = NPAD // RB


def _proj1_body(x_ref, wl_ref, wr_ref, xl_ref, xr_ref):
    x = x_ref[...]
    xl_ref[...] = jnp.dot(x, wl_ref[...], preferred_element_type=jnp.float32)
    xr_ref[...] = jnp.dot(x, wr_ref[...], preferred_element_type=jnp.float32)


def _proj1(xpad, Wl1, Wr1):
    return pl.pallas_call(
        _proj1_body,
        grid=(NRB, 2),
        in_specs=[
            pl.BlockSpec((RB, F_IN), lambda i, h: (i, 0)),
            pl.BlockSpec((F_IN, HID), lambda i, h: (0, h)),
            pl.BlockSpec((F_IN, HID), lambda i, h: (0, h)),
        ],
        out_specs=[
            pl.BlockSpec((RB, HID), lambda i, h: (h * NRB + i, 0)),
            pl.BlockSpec((RB, HID), lambda i, h: (h * NRB + i, 0)),
        ],
        out_shape=[
            jax.ShapeDtypeStruct((2 * NPAD, HID), jnp.float32),
            jax.ShapeDtypeStruct((2 * NPAD, HID), jnp.float32),
        ],
    )(xpad, Wl1, Wr1)


def _elu(v):
    return jnp.where(v > 0, v, jnp.exp(v) - 1.0)


def _mid_body(h_ref, d_ref, b1_ref, wl_ref, wr_ref, xl_ref, xr_ref):
    e0 = _elu(h_ref[0] / (d_ref[0] + 1e-16) + b1_ref[0])
    e1 = _elu(h_ref[1] / (d_ref[1] + 1e-16) + b1_ref[1])
    wl = wl_ref[...]
    wr = wr_ref[...]
    xl_ref[...] = (jnp.dot(e0, wl[:HID], preferred_element_type=jnp.float32)
                   + jnp.dot(e1, wl[HID:], preferred_element_type=jnp.float32))
    xr_ref[...] = (jnp.dot(e0, wr[:HID], preferred_element_type=jnp.float32)
                   + jnp.dot(e1, wr[HID:], preferred_element_type=jnp.float32))


def _mid(h1r, d1r, b1r, Wl2, Wr2):
    return pl.pallas_call(
        _mid_body,
        grid=(NRB,),
        in_specs=[
            pl.BlockSpec((2, RB, HID), lambda i: (0, i, 0)),
            pl.BlockSpec((2, RB, 1), lambda i: (0, i, 0)),
            pl.BlockSpec((2, HID), lambda i: (0, 0)),
            pl.BlockSpec((2 * HID, HID), lambda i: (0, 0)),
            pl.BlockSpec((2 * HID, HID), lambda i: (0, 0)),
        ],
        out_specs=[
            pl.BlockSpec((RB, HID), lambda i: (i, 0)),
            pl.BlockSpec((RB, HID), lambda i: (i, 0)),
        ],
        out_shape=[
            jax.ShapeDtypeStruct((NPAD, HID), jnp.float32),
            jax.ShapeDtypeStruct((NPAD, HID), jnp.float32),
        ],
    )(h1r, d1r, b1r, Wl2, Wr2)


def _final_body(p_ref, d_ref, b2_ref, batch_ref, wlin_ref, blin_ref, out_ref,
                sum_scr, cnt_scr):
    i = pl.program_id(0)

    @pl.when(i == 0)
    def _():
        sum_scr[...] = jnp.zeros((NG, HID), jnp.float32)
        cnt_scr[...] = jnp.zeros((NG, HID), jnp.float32)

    q = (p_ref[0] + p_ref[1]) / (d_ref[0] + d_ref[1] + 1e-16)
    h = _elu(q + b2_ref[0])
    b = batch_ref[0, 0, :]
    P = (lax.broadcasted_iota(jnp.int32, (NG, RB), 0) == b[None, :]
         ).astype(jnp.float32)
    sum_scr[...] += jnp.dot(P, h, preferred_element_type=jnp.float32)
    cnt_scr[...] += jnp.dot(P, jnp.ones((RB, HID), jnp.float32),
                            preferred_element_type=jnp.float32)

    @pl.when(i == NRB - 1)
    def _():
        pooled = sum_scr[...] / jnp.maximum(cnt_scr[...], 1.0)
        out_ref[...] = (jnp.dot(pooled, wlin_ref[...],
                                preferred_element_type=jnp.float32)
                        + blin_ref[0])


def _final(p2r, d2r, b2r, batch3, wlin_p, blin_p):
    return pl.pallas_call(
        _final_body,
        grid=(NRB,),
        in_specs=[
            pl.BlockSpec((2, RB, HID), lambda i: (0, i, 0)),
            pl.BlockSpec((2, RB, 1), lambda i: (0, i, 0)),
            pl.BlockSpec((1, HID), lambda i: (0, 0)),
            pl.BlockSpec((1, 1, RB), lambda i: (i, 0, 0)),
            pl.BlockSpec((HID, 128), lambda i: (0, 0)),
            pl.BlockSpec((1, 128), lambda i: (0, 0)),
        ],
        out_specs=pl.BlockSpec((NG, 128), lambda i: (0, 0)),
        out_shape=jax.ShapeDtypeStruct((NG, 128), jnp.float32),
        scratch_shapes=[
            pltpu.VMEM((NG, HID), jnp.float32),
            pltpu.VMEM((NG, HID), jnp.float32),
        ],
    )(p2r, d2r, b2r, batch3, wlin_p, blin_p)


_MESH = plsc.VectorSubcoreMesh(core_axis_name="c", subcore_axis_name="s")

_BUFSET = [
    pltpu.VMEM((EB,), jnp.int32),
    pltpu.VMEM((EB,), jnp.int32),
    pltpu.VMEM((EB,), jnp.int32),
    pltpu.VMEM((EB,), jnp.int32),
    pltpu.VMEM((EB, HID), jnp.float32),
    pltpu.VMEM((EB, HID), jnp.float32),
    pltpu.VMEM((EB,), jnp.float32),
    pltpu.VMEM((EB + 16,), jnp.float32),
    pltpu.VMEM((EB,), jnp.int32),
]
_SC_SCRATCH = (_BUFSET + _BUFSET + [
    pltpu.VMEM((HID,), jnp.float32),
    pltpu.VMEM((EB, 17), jnp.float32),
] + [pltpu.SemaphoreType.DMA] * 12
  + [
    pltpu.VMEM_SHARED((NPAD,), jnp.float32),
    pltpu.VMEM_SHARED((NPAD, HID), jnp.float32),
])

_SC_OUT = [jax.ShapeDtypeStruct((2 * NPAD, HID), jnp.float32),
           jax.ShapeDtypeStruct((2 * NPAD,), jnp.float32)]


def _edge_block_logits(xls_v, xrd_v, att_vecs, ebuf, e_v):
    @plsc.parallel_loop(0, EB, 1, unroll=8)
    def edge_body(je):
        acc = jnp.zeros((16,), jnp.float32)
        for k in range(HID // 16):
            a = xls_v[je, pl.ds(16 * k, 16)]
            b = xrd_v[je, pl.ds(16 * k, 16)]
            h = a + b
            h = jnp.where(h > 0, h, 0.2 * h)
            acc = acc + h * att_vecs[k]
        ebuf[je, pl.ds(0, 16)] = acc

    iota = lax.broadcasted_iota(jnp.int32, (16,), 0)
    for g in range(EB // 16):
        rows = iota + (16 * g)
        tot = jnp.zeros((16,), jnp.float32)
        for l in range(16):
            tot = tot + plsc.load_gather(
                ebuf, [rows, jnp.full((16,), l, jnp.int32)])
        e_v[pl.ds(16 * g, 16)] = jnp.exp(tot)


def _scale_rows(xls_v, a_v):
    @plsc.parallel_loop(0, EB, 1, unroll=8)
    def body(je):
        a = a_v[pl.ds(je, 16)][0]
        for k in range(HID // 16):
            xls_v[je, pl.ds(16 * k, 16)] = xls_v[je, pl.ds(16 * k, 16)] * a


def _sc_prologue(z1_hbm, z2_hbm, att_hbm, att_row, att_v, denom_sh, out_sh, s):
    r0 = s * ROWS_T
    pltpu.sync_copy(z1_hbm.at[pl.ds(r0, ROWS_T)],
                    denom_sh.at[pl.ds(r0, ROWS_T)])
    pltpu.sync_copy(z2_hbm.at[pl.ds(r0, ROWS_T)],
                    out_sh.at[pl.ds(r0, ROWS_T)])
    pltpu.sync_copy(att_hbm.at[att_row], att_v)
    plsc.subcore_barrier()
    return [att_v[pl.ds(16 * k, 16)] for k in range(HID // 16)]


def _sc_pipeline(nblk_t, base_of, noff, with_off, xl_hbm, xr_hbm, src_hbm,
                 dst_hbm, att_vecs, bufs, isems, gsems, ssems, ebuf,
                 denom_sh, out_sh):
    def issue_idx(bi, p):
        base = base_of(bi)
        pltpu.async_copy(src_hbm.at[pl.ds(base, EB)], bufs[p][0],
                         isems[p][0])
        pltpu.async_copy(dst_hbm.at[pl.ds(base, EB)], bufs[p][1],
                         isems[p][1])

    def wait_idx(p):
        pltpu.make_async_copy(src_hbm.at[pl.ds(0, EB)], bufs[p][0],
                              isems[p][0]).wait()
        pltpu.make_async_copy(dst_hbm.at[pl.ds(0, EB)], bufs[p][1],
                              isems[p][1]).wait()

    def issue_gathers(p):
        src_v, dst_v, gidx_v, gidx2_v, xls_v, xrd_v = bufs[p][:6]
        if with_off:
            for g in range(EB // 16):
                gidx_v[pl.ds(16 * g, 16)] = src_v[pl.ds(16 * g, 16)] + noff
                gidx2_v[pl.ds(16 * g, 16)] = dst_v[pl.ds(16 * g, 16)] + noff
            pltpu.async_copy(xl_hbm.at[gidx_v], xls_v, gsems[p][0])
            pltpu.async_copy(xr_hbm.at[gidx2_v], xrd_v, gsems[p][1])
        else:
            pltpu.async_copy(xl_hbm.at[src_v], xls_v, gsems[p][0])
            pltpu.async_copy(xr_hbm.at[dst_v], xrd_v, gsems[p][1])

    def wait_gathers(p):
        pltpu.make_async_copy(xl_hbm.at[pl.ds(0, EB)], bufs[p][4],
                              gsems[p][0]).wait()
        pltpu.make_async_copy(xl_hbm.at[pl.ds(0, EB)], bufs[p][5],
                              gsems[p][1]).wait()

    def wait_scatters(p):
        pltpu.make_async_copy(bufs[p][6], denom_sh.at[pl.ds(0, EB)],
                              ssems[p][0]).wait()
        pltpu.make_async_copy(bufs[p][4], out_sh.at[pl.ds(0, EB)],
                              ssems[p][1]).wait()

    issue_idx(0, 0)
    wait_idx(0)
    issue_gathers(0)
    issue_idx(1, 1)

    def pair(j, carry):
        for p in (0, 1):
            q = 1 - p
            i = 2 * j + p
            wait_idx(q)
            if p == 0:
                @pl.when(j > 0)
                def _():
                    wait_scatters(q)
            else:
                wait_scatters(q)
            issue_gathers(q)
            wait_gathers(p)
            dst_v = bufs[p][1]
            xls_v = bufs[p][4]
            e_v = bufs[p][6]
            a_v = bufs[p][7]
            dscat_v = bufs[p][8]
            _edge_block_logits(xls_v, bufs[p][5], att_vecs, ebuf, e_v)
            for g in range(EB // 16):
                a_v[pl.ds(16 * g, 16)] = e_v[pl.ds(16 * g, 16)]
                dscat_v[pl.ds(16 * g, 16)] = dst_v[pl.ds(16 * g, 16)]
            _scale_rows(xls_v, a_v)
            pltpu.async_copy(e_v, denom_sh.at[dscat_v], ssems[p][0],
                             add=True)
            pltpu.async_copy(xls_v, out_sh.at[dscat_v], ssems[p][1],
                             add=True)
            issue_idx(lax.rem(i + 2, nblk_t), p)
        return carry

    lax.fori_loop(0, nblk_t // 2, pair, 0)
    wait_scatters(1)
    wait_gathers(0)
    wait_idx(1)


def _sc_epilogue(out_hbm, d_hbm, denom_sh, out_sh, row_off, s):
    plsc.subcore_barrier()
    r0 = s * ROWS_T
    pltpu.sync_copy(out_sh.at[pl.ds(r0, ROWS_T)],
                    out_hbm.at[pl.ds(row_off + r0, ROWS_T)])
    pltpu.sync_copy(denom_sh.at[pl.ds(r0, ROWS_T)],
                    d_hbm.at[pl.ds(row_off + r0, ROWS_T)])


@functools.partial(
    pl.kernel,
    out_type=_SC_OUT,
    mesh=_MESH,
    compiler_params=pltpu.CompilerParams(needs_layout_passes=False),
    scratch_types=_SC_SCRATCH,
)
def _gat1_sc(xl_hbm, xr_hbm, src_hbm, dst_hbm, att_hbm, z1_hbm, z2_hbm,
             out_hbm, d_hbm,
             b0_src, b0_dst, b0_gi, b0_gi2, b0_xls, b0_xrd, b0_e, b0_a,
             b0_ds, b1_src, b1_dst, b1_gi, b1_gi2, b1_xls, b1_xrd, b1_e,
             b1_a, b1_ds, att_v, ebuf, i_a0, i_b0, i_a1, i_b1, g_a0, g_b0,
             g_a1, g_b1, s_a0, s_b0, s_a1, s_b1, denom_sh, out_sh):
    c = lax.axis_index("c")
    s = lax.axis_index("s")
    noff = c * NPAD
    att_vecs = _sc_prologue(z1_hbm, z2_hbm, att_hbm, c, att_v, denom_sh,
                            out_sh, s)
    bufs = ((b0_src, b0_dst, b0_gi, b0_gi2, b0_xls, b0_xrd, b0_e, b0_a,
             b0_ds),
            (b1_src, b1_dst, b1_gi, b1_gi2, b1_xls, b1_xrd, b1_e, b1_a,
             b1_ds))
    blk0 = s * BLK_T1
    _sc_pipeline(BLK_T1, lambda bi: (blk0 + bi) * EB, noff, True, xl_hbm,
                 xr_hbm, src_hbm, dst_hbm, att_vecs, bufs,
                 ((i_a0, i_b0), (i_a1, i_b1)), ((g_a0, g_b0), (g_a1, g_b1)),
                 ((s_a0, s_b0), (s_a1, s_b1)), ebuf, denom_sh, out_sh)
    _sc_epilogue(out_hbm, d_hbm, denom_sh, out_sh, noff, s)


@functools.partial(
    pl.kernel,
    out_type=_SC_OUT,
    mesh=_MESH,
    compiler_params=pltpu.CompilerParams(needs_layout_passes=False),
    scratch_types=_SC_SCRATCH,
)
def _gat2_sc(xl_hbm, xr_hbm, src_hbm, dst_hbm, att_hbm, z1_hbm, z2_hbm,
             out_hbm, d_hbm,
             b0_src, b0_dst, b0_gi, b0_gi2, b0_xls, b0_xrd, b0_e, b0_a,
             b0_ds, b1_src, b1_dst, b1_gi, b1_gi2, b1_xls, b1_xrd, b1_e,
             b1_a, b1_ds, att_v, ebuf, i_a0, i_b0, i_a1, i_b1, g_a0, g_b0,
             g_a1, g_b1, s_a0, s_b0, s_a1, s_b1, denom_sh, out_sh):
    c = lax.axis_index("c")
    s = lax.axis_index("s")
    att_vecs = _sc_prologue(z1_hbm, z2_hbm, att_hbm, 0, att_v, denom_sh,
                            out_sh, s)
    bufs = ((b0_src, b0_dst, b0_gi, b0_gi2, b0_xls, b0_xrd, b0_e, b0_a,
             b0_ds),
            (b1_src, b1_dst, b1_gi, b1_gi2, b1_xls, b1_xrd, b1_e, b1_a,
             b1_ds))
    my_blk0 = (c * 16 + s) * BLK_W2
    _sc_pipeline(BLK_W2, lambda bi: (my_blk0 + bi) * EB, 0, False, xl_hbm,
                 xr_hbm, src_hbm, dst_hbm, att_vecs, bufs,
                 ((i_a0, i_b0), (i_a1, i_b1)), ((g_a0, g_b0), (g_a1, g_b1)),
                 ((s_a0, s_b0), (s_a1, s_b1)), ebuf, denom_sh, out_sh)
    _sc_epilogue(out_hbm, d_hbm, denom_sh, out_sh, c * NPAD, s)


def kernel(x, edge_index, batch, Wl1, Wr1, att1, b1, Wl2, Wr2, att2, b2,
           Wlin, blin):
    f32 = jnp.float32
    i32 = jnp.int32
    npad_e = EPAD - E_TOT
    loops = jnp.arange(N, dtype=i32)
    pad_src = jnp.arange(npad_e, dtype=i32) % N
    pad_dst = N + jnp.arange(npad_e, dtype=i32) % (NPAD - N)
    src = jnp.concatenate([edge_index[0].astype(i32), loops, pad_src])
    dst = jnp.concatenate([edge_index[1].astype(i32), loops, pad_dst])

    xpad = jnp.pad(x.astype(f32), ((0, NPAD - N), (0, 0)))
    z1 = jnp.zeros((NPAD,), f32)
    z2 = jnp.zeros((NPAD, HID), f32)

    xl1, xr1 = _proj1(xpad, Wl1, Wr1)
    h1, d1 = _gat1_sc(xl1, xr1, src, dst, att1, z1, z2)
    xl2, xr2 = _mid(h1.reshape(2, NPAD, HID), d1.reshape(2, NPAD, 1),
                    b1.reshape(2, HID), Wl2, Wr2)
    h2, d2 = _gat2_sc(xl2, xr2, src, dst, att2, z1, z2)

    batch3 = jnp.concatenate(
        [batch.astype(i32), jnp.full((NPAD - N,), NG, i32)]).reshape(NRB, 1, RB)
    wlin_p = jnp.pad(Wlin.astype(f32), ((0, 0), (0, 128 - NC)))
    blin_p = jnp.pad(blin.astype(f32), (0, 128 - NC)).reshape(1, 128)
    logits = _final(h2.reshape(2, NPAD, HID), d2.reshape(2, NPAD, 1),
                    b2.reshape(1, HID), batch3, wlin_p, blin_p)
    return logits[:, :NC]
